# Initial kernel scaffold; baseline (speedup 1.0000x reference)
import functools

import jax
import jax.numpy as jnp
from jax import lax
from jax.experimental import pallas as pl
from jax.experimental.pallas import tpu as pltpu
from jax.experimental.pallas import tpu_sc as plsc

N = 10000
E = 320000
D_IN = 128
D_HID = 128
N_CLASSES = 64

NC = 2
NS = 16
NW = NC * NS
EPT = E // NW
CH = 80
NCHUNK = EPT // CH
RPT = N // NS
NDEG = 10
DPT = N // NDEG

_MESH = plsc.VectorSubcoreMesh(core_axis_name="c", subcore_axis_name="s")


def _sc_agg0_body(h_hbm, src_hbm, dst_hbm, z2d_hbm, z1d_hbm,
                  acc_out, deg_out,
                  srcv, dstv, rows, ones, acc_sh, deg_sh, sem):
    cid = lax.axis_index("c")
    sid = lax.axis_index("s")
    wid = cid * NS + sid

    pltpu.sync_copy(z2d_hbm.at[pl.ds(sid * RPT, RPT)],
                    acc_sh.at[pl.ds(sid * RPT, RPT)])

    @pl.when(sid < NDEG)
    def _():
        pltpu.sync_copy(z1d_hbm.at[pl.ds(sid * DPT, DPT)],
                        deg_sh.at[pl.ds(sid * DPT, DPT)])

    pltpu.sync_copy(src_hbm.at[wid], srcv)
    pltpu.sync_copy(dst_hbm.at[wid], dstv)
    for k in range(CH // 16):
        ones[pl.ds(k * 16, 16)] = jnp.full((16,), 1.0, jnp.float32)

    plsc.subcore_barrier()

    def step(i, carry):
        pltpu.async_copy(h_hbm.at[srcv.at[i]], rows, sem).wait()
        pltpu.sync_copy(rows, acc_sh.at[dstv.at[i]], add=True)
        pltpu.sync_copy(ones, deg_sh.at[dstv.at[i]], add=True)
        return carry

    lax.fori_loop(0, NCHUNK, step, 0)

    plsc.subcore_barrier()

    pltpu.sync_copy(acc_sh.at[pl.ds(sid * RPT, RPT)],
                    acc_out.at[cid, pl.ds(sid * RPT, RPT)])

    @pl.when(sid < NDEG)
    def _():
        pltpu.sync_copy(deg_sh.at[pl.ds(sid * DPT, DPT)],
                        deg_out.at[cid, pl.ds(sid * DPT, DPT)])


_sc_agg0 = functools.partial(
    pl.kernel,
    out_type=[jax.ShapeDtypeStruct((NC, N, D_HID), jnp.float32),
              jax.ShapeDtypeStruct((NC, N), jnp.float32)],
    mesh=_MESH,
    scratch_types=[
        pltpu.VMEM((NCHUNK, CH), jnp.int32),
        pltpu.VMEM((NCHUNK, CH), jnp.int32),
        pltpu.VMEM((CH, D_HID), jnp.float32),
        pltpu.VMEM((CH,), jnp.float32),
        pltpu.VMEM_SHARED((N, D_HID), jnp.float32),
        pltpu.VMEM_SHARED((N,), jnp.float32),
        pltpu.SemaphoreType.DMA,
    ],
)(_sc_agg0_body)


def _sc_agg1_body(h_hbm, src_hbm, dst_hbm, z2d_hbm,
                  acc_out,
                  srcv, dstv, rows, acc_sh, sem):
    cid = lax.axis_index("c")
    sid = lax.axis_index("s")
    wid = cid * NS + sid

    pltpu.sync_copy(z2d_hbm.at[pl.ds(sid * RPT, RPT)],
                    acc_sh.at[pl.ds(sid * RPT, RPT)])
    pltpu.sync_copy(src_hbm.at[wid], srcv)
    pltpu.sync_copy(dst_hbm.at[wid], dstv)

    plsc.subcore_barrier()

    def step(i, carry):
        pltpu.async_copy(h_hbm.at[srcv.at[i]], rows, sem).wait()
        pltpu.sync_copy(rows, acc_sh.at[dstv.at[i]], add=True)
        return carry

    lax.fori_loop(0, NCHUNK, step, 0)

    plsc.subcore_barrier()

    pltpu.sync_copy(acc_sh.at[pl.ds(sid * RPT, RPT)],
                    acc_out.at[cid, pl.ds(sid * RPT, RPT)])


_sc_agg1 = functools.partial(
    pl.kernel,
    out_type=jax.ShapeDtypeStruct((NC, N, N_CLASSES), jnp.float32),
    mesh=_MESH,
    scratch_types=[
        pltpu.VMEM((NCHUNK, CH), jnp.int32),
        pltpu.VMEM((NCHUNK, CH), jnp.int32),
        pltpu.VMEM((CH, N_CLASSES), jnp.float32),
        pltpu.VMEM_SHARED((N, N_CLASSES), jnp.float32),
        pltpu.SemaphoreType.DMA,
    ],
)(_sc_agg1_body)


def _tc1_body(x_ref, acc_ref, deg_ref, ws0_ref, wn0_ref, b0_ref,
              g0_ref, be0_ref, ws1_ref, wn1_ref, b1_ref,
              z1_ref, s1_ref, rd_ref):
    rd = 1.0 / jnp.maximum(deg_ref[0] + deg_ref[1], 1.0)
    hn = (acc_ref[0] + acc_ref[1]) * rd
    h = (jnp.dot(x_ref[...], ws0_ref[...],
                 preferred_element_type=jnp.float32)
         + jnp.dot(hn, wn0_ref[...], preferred_element_type=jnp.float32)
         + b0_ref[...])
    mu = jnp.mean(h, axis=0, keepdims=True)
    var = jnp.mean(jnp.square(h - mu), axis=0, keepdims=True)
    h = g0_ref[...] * (h - mu) * lax.rsqrt(var + 1e-5) + be0_ref[...]
    h = jnp.maximum(h, 0.0)
    z1_ref[...] = jnp.dot(h, wn1_ref[...], preferred_element_type=jnp.float32)
    s1_ref[...] = (jnp.dot(h, ws1_ref[...], preferred_element_type=jnp.float32)
                   + b1_ref[...])
    rd_ref[...] = rd


def _tc2_body(s1_ref, acc_ref, rd_ref, out_ref):
    out_ref[...] = s1_ref[...] + (acc_ref[0] + acc_ref[1]) * rd_ref[...]


def kernel(x, edge_index, W_self0, W_neigh0, b0, gamma0, beta0,
           W_self1, W_neigh1, b1):
    src3 = edge_index[0].reshape(NW, NCHUNK, CH)
    dst3 = edge_index[1].reshape(NW, NCHUNK, CH)
    z2d = jnp.zeros((N, D_HID), jnp.float32)
    z1d = jnp.zeros((N,), jnp.float32)

    acc0, deg = _sc_agg0(x, src3, dst3, z2d, z1d)

    z1, s1, rd = pl.pallas_call(
        _tc1_body,
        out_shape=[jax.ShapeDtypeStruct((N, N_CLASSES), jnp.float32),
                   jax.ShapeDtypeStruct((N, N_CLASSES), jnp.float32),
                   jax.ShapeDtypeStruct((N, 1), jnp.float32)],
    )(x, acc0, deg.reshape(NC, N, 1), W_self0, W_neigh0,
      b0.reshape(1, -1), gamma0.reshape(1, -1), beta0.reshape(1, -1),
      W_self1, W_neigh1, b1.reshape(1, -1))

    acc1 = _sc_agg1(z1, src3, dst3, jnp.zeros((N, N_CLASSES), jnp.float32))

    out = pl.pallas_call(
        _tc2_body,
        out_shape=jax.ShapeDtypeStruct((N, N_CLASSES), jnp.float32),
    )(s1, acc1, rd)
    return out


# speedup vs baseline: 8.2893x; 8.2893x over previous
import functools

import jax
import jax.numpy as jnp
from jax import lax
from jax.experimental import pallas as pl
from jax.experimental.pallas import tpu as pltpu
from jax.experimental.pallas import tpu_sc as plsc

N = 10000
E = 320000
D_IN = 128
D_HID = 128
N_CLASSES = 64

NC = 2
NS = 16
NW = NC * NS
EPT = E // NW
CH = 80
NCHUNK = EPT // CH
RPT = 624
RPT_LAST = N - 15 * RPT
NDEG = 5
DPT = N // NDEG

_MESH = plsc.VectorSubcoreMesh(core_axis_name="c", subcore_axis_name="s")


def _zero_init_rows(z_hbm, acc_sh, sid):
    @pl.when(sid < NS - 1)
    def _():
        s = pl.ds(sid * RPT, RPT)
        pltpu.sync_copy(z_hbm.at[s], acc_sh.at[s])

    @pl.when(sid == NS - 1)
    def _():
        s = pl.ds((NS - 1) * RPT, RPT_LAST)
        pltpu.sync_copy(z_hbm.at[s], acc_sh.at[s])


def _copy_out_rows(acc_sh, acc_out, cid, sid):
    @pl.when(sid < NS - 1)
    def _():
        pltpu.sync_copy(acc_sh.at[pl.ds(sid * RPT, RPT)],
                        acc_out.at[cid, pl.ds(sid * RPT, RPT)])

    @pl.when(sid == NS - 1)
    def _():
        pltpu.sync_copy(acc_sh.at[pl.ds((NS - 1) * RPT, RPT_LAST)],
                        acc_out.at[cid, pl.ds((NS - 1) * RPT, RPT_LAST)])


def _sc_agg0_body(h_hbm, src_hbm, dst_hbm, z2d_hbm,
                  acc_out, deg_out,
                  srcv, dstv, rows, ones, degv, acc_sh, deg_sh, sem):
    cid = lax.axis_index("c")
    sid = lax.axis_index("s")
    wid = cid * NS + sid

    _zero_init_rows(z2d_hbm, acc_sh, sid)

    def fill_zero(i, carry):
        degv[pl.ds(i * 16, 16)] = jnp.zeros((16,), jnp.float32)
        return carry

    lax.fori_loop(0, DPT // 16, fill_zero, 0)

    @pl.when(sid < NDEG)
    def _():
        pltpu.sync_copy(degv, deg_sh.at[pl.ds(sid * DPT, DPT)])

    pltpu.sync_copy(src_hbm.at[wid], srcv)
    pltpu.sync_copy(dst_hbm.at[wid], dstv)
    for k in range(CH // 16):
        ones[pl.ds(k * 16, 16)] = jnp.full((16,), 1.0, jnp.float32)

    plsc.subcore_barrier()

    def step(i, carry):
        pltpu.async_copy(h_hbm.at[srcv.at[i]], rows, sem).wait()
        pltpu.sync_copy(rows, acc_sh.at[dstv.at[i]], add=True)
        pltpu.sync_copy(ones, deg_sh.at[dstv.at[i]], add=True)
        return carry

    lax.fori_loop(0, NCHUNK, step, 0)

    plsc.subcore_barrier()

    _copy_out_rows(acc_sh, acc_out, cid, sid)

    @pl.when(sid < NDEG)
    def _():
        pltpu.sync_copy(deg_sh.at[pl.ds(sid * DPT, DPT)], degv)
        pltpu.sync_copy(degv, deg_out.at[pl.ds(cid * N + sid * DPT, DPT)])


_sc_agg0 = functools.partial(
    pl.kernel,
    out_type=[jax.ShapeDtypeStruct((NC, N, D_HID), jnp.float32),
              jax.ShapeDtypeStruct((NC * N,), jnp.float32)],
    mesh=_MESH,
    scratch_types=[
        pltpu.VMEM((NCHUNK, CH), jnp.int32),
        pltpu.VMEM((NCHUNK, CH), jnp.int32),
        pltpu.VMEM((CH, D_HID), jnp.float32),
        pltpu.VMEM((CH,), jnp.float32),
        pltpu.VMEM((DPT,), jnp.float32),
        pltpu.VMEM_SHARED((N, D_HID), jnp.float32),
        pltpu.VMEM_SHARED((N,), jnp.float32),
        pltpu.SemaphoreType.DMA,
    ],
)(_sc_agg0_body)


def _sc_agg1_body(h_hbm, src_hbm, dst_hbm, z2d_hbm,
                  acc_out,
                  srcv, dstv, rows, acc_sh, sem):
    cid = lax.axis_index("c")
    sid = lax.axis_index("s")
    wid = cid * NS + sid

    _zero_init_rows(z2d_hbm, acc_sh, sid)
    pltpu.sync_copy(src_hbm.at[wid], srcv)
    pltpu.sync_copy(dst_hbm.at[wid], dstv)

    plsc.subcore_barrier()

    def step(i, carry):
        pltpu.async_copy(h_hbm.at[srcv.at[i]], rows, sem).wait()
        pltpu.sync_copy(rows, acc_sh.at[dstv.at[i]], add=True)
        return carry

    lax.fori_loop(0, NCHUNK, step, 0)

    plsc.subcore_barrier()

    _copy_out_rows(acc_sh, acc_out, cid, sid)


_sc_agg1 = functools.partial(
    pl.kernel,
    out_type=jax.ShapeDtypeStruct((NC, N, N_CLASSES), jnp.float32),
    mesh=_MESH,
    scratch_types=[
        pltpu.VMEM((NCHUNK, CH), jnp.int32),
        pltpu.VMEM((NCHUNK, CH), jnp.int32),
        pltpu.VMEM((CH, N_CLASSES), jnp.float32),
        pltpu.VMEM_SHARED((N, N_CLASSES), jnp.float32),
        pltpu.SemaphoreType.DMA,
    ],
    compiler_params=pltpu.CompilerParams(use_tc_tiling_on_sc=False),
)(_sc_agg1_body)


def _tc1_body(x_ref, acc_ref, deg_ref, ws0_ref, wn0_ref, b0_ref,
              g0_ref, be0_ref, ws1_ref, wn1_ref, b1_ref,
              z1_ref, s1_ref, rd_ref):
    rd = 1.0 / jnp.maximum(deg_ref[0] + deg_ref[1], 1.0)
    hn = (acc_ref[0] + acc_ref[1]) * rd
    h = (jnp.dot(x_ref[...], ws0_ref[...],
                 preferred_element_type=jnp.float32)
         + jnp.dot(hn, wn0_ref[...], preferred_element_type=jnp.float32)
         + b0_ref[...])
    mu = jnp.mean(h, axis=0, keepdims=True)
    var = jnp.mean(jnp.square(h - mu), axis=0, keepdims=True)
    h = g0_ref[...] * (h - mu) * lax.rsqrt(var + 1e-5) + be0_ref[...]
    h = jnp.maximum(h, 0.0)
    z1_ref[...] = jnp.dot(h, wn1_ref[...], preferred_element_type=jnp.float32)
    s1_ref[...] = (jnp.dot(h, ws1_ref[...], preferred_element_type=jnp.float32)
                   + b1_ref[...])
    rd_ref[...] = rd


def _tc2_body(s1_ref, acc_ref, rd_ref, out_ref):
    out_ref[...] = s1_ref[...] + (acc_ref[0] + acc_ref[1]) * rd_ref[...]


def kernel(x, edge_index, W_self0, W_neigh0, b0, gamma0, beta0,
           W_self1, W_neigh1, b1):
    src3 = edge_index[0].reshape(NW, NCHUNK, CH)
    dst3 = edge_index[1].reshape(NW, NCHUNK, CH)
    z2d = jnp.zeros((N, D_HID), jnp.float32)

    acc0, deg = _sc_agg0(x, src3, dst3, z2d)

    z1, s1, rd = pl.pallas_call(
        _tc1_body,
        out_shape=[jax.ShapeDtypeStruct((N, N_CLASSES), jnp.float32),
                   jax.ShapeDtypeStruct((N, N_CLASSES), jnp.float32),
                   jax.ShapeDtypeStruct((N, 1), jnp.float32)],
    )(x, acc0, deg.reshape(NC, N, 1), W_self0, W_neigh0,
      b0.reshape(1, -1), gamma0.reshape(1, -1), beta0.reshape(1, -1),
      W_self1, W_neigh1, b1.reshape(1, -1))

    acc1 = _sc_agg1(z1, src3, dst3, jnp.zeros((N, N_CLASSES), jnp.float32))

    out = pl.pallas_call(
        _tc2_body,
        out_shape=jax.ShapeDtypeStruct((N, N_CLASSES), jnp.float32),
    )(s1, acc1, rd)
    return out


# speedup vs baseline: 12.7822x; 1.5420x over previous
import functools

import jax
import jax.numpy as jnp
from jax import lax
from jax.experimental import pallas as pl
from jax.experimental.pallas import tpu as pltpu
from jax.experimental.pallas import tpu_sc as plsc

N = 10000
E = 320000
D_IN = 128
D_HID = 128
N_CLASSES = 64

NC = 2
NS = 16
NW = NC * NS
EPT = E // NW
CH0 = 80
NCHUNK0 = EPT // CH0
CH1 = 80
NCHUNK1 = EPT // CH1
RPT = 624
RPT_LAST = N - 15 * RPT
NDEG = 5
DPT = N // NDEG

_MESH = plsc.VectorSubcoreMesh(core_axis_name="c", subcore_axis_name="s")


def _zero_init_rows(z_hbm, acc_sh, sid):
    @pl.when(sid < NS - 1)
    def _():
        s = pl.ds(sid * RPT, RPT)
        pltpu.sync_copy(z_hbm.at[s], acc_sh.at[s])

    @pl.when(sid == NS - 1)
    def _():
        s = pl.ds((NS - 1) * RPT, RPT_LAST)
        pltpu.sync_copy(z_hbm.at[s], acc_sh.at[s])


def _copy_out_rows(acc_sh, acc_out, cid, sid):
    @pl.when(sid < NS - 1)
    def _():
        pltpu.sync_copy(acc_sh.at[pl.ds(sid * RPT, RPT)],
                        acc_out.at[cid, pl.ds(sid * RPT, RPT)])

    @pl.when(sid == NS - 1)
    def _():
        pltpu.sync_copy(acc_sh.at[pl.ds((NS - 1) * RPT, RPT_LAST)],
                        acc_out.at[cid, pl.ds((NS - 1) * RPT, RPT_LAST)])


def _agg_pipeline(nchunk, ch, h_hbm, srcv, dstv, acc_sh, rows0, rows1,
                  g0, g1, s0, s1, deg_sh=None, ones=None, dsem=None):
    def gather(i, buf, sem):
        pltpu.async_copy(h_hbm.at[srcv.at[pl.ds(i * ch, ch)]], buf, sem)

    def wait_g(buf, sem):
        pltpu.make_async_copy(h_hbm.at[srcv.at[pl.ds(0, ch)]], buf,
                              sem).wait()

    def scat(i, buf, sem):
        pltpu.async_copy(buf, acc_sh.at[dstv.at[i]], sem, add=True)

    def wait_s(buf, sem):
        pltpu.make_async_copy(buf, acc_sh.at[dstv.at[0]], sem).wait()

    def deg(i):
        if deg_sh is not None:
            if not (isinstance(i, int) and i < 4):
                @pl.when(i >= 4)
                def _():
                    pltpu.make_async_copy(ones, deg_sh.at[dstv.at[0]],
                                          dsem).wait()
            pltpu.async_copy(ones, deg_sh.at[dstv.at[i]], dsem, add=True)

    pairs = (nchunk - 1) // 2

    gather(0, rows0, g0)
    gather(1, rows1, g1)
    wait_g(rows0, g0)
    deg(0)
    scat(0, rows0, s0)

    def body(j, carry):
        a = 2 * j + 1
        wait_s(rows0, s0)
        gather(a + 1, rows0, g0)
        wait_g(rows1, g1)
        deg(a)
        scat(a, rows1, s1)
        wait_s(rows1, s1)

        @pl.when(j < pairs - 1)
        def _():
            gather(a + 2, rows1, g1)

        wait_g(rows0, g0)
        deg(a + 1)
        scat(a + 1, rows0, s0)
        return carry

    lax.fori_loop(0, pairs, body, 0)
    wait_s(rows0, s0)

    if deg_sh is not None:
        for _ in range(min(4, nchunk)):
            pltpu.make_async_copy(ones, deg_sh.at[dstv.at[0]], dsem).wait()


def _sc_agg0_body(h_hbm, src_hbm, dst_hbm, z2d_hbm,
                  acc_out, deg_out,
                  srcv, dstv, rows0, rows1, ones, degv, acc_sh, deg_sh,
                  g0, g1, s0, s1, dsem):
    cid = lax.axis_index("c")
    sid = lax.axis_index("s")
    wid = cid * NS + sid

    _zero_init_rows(z2d_hbm, acc_sh, sid)

    def fill_zero(i, carry):
        degv[pl.ds(i * 16, 16)] = jnp.zeros((16,), jnp.float32)
        return carry

    lax.fori_loop(0, DPT // 16, fill_zero, 0)

    @pl.when(sid < NDEG)
    def _():
        pltpu.sync_copy(degv, deg_sh.at[pl.ds(sid * DPT, DPT)])

    pltpu.sync_copy(src_hbm.at[wid], srcv)
    pltpu.sync_copy(dst_hbm.at[wid], dstv)
    one_offs = list(range(0, CH0 - 15, 16))
    if CH0 % 16:
        one_offs.append(CH0 - 16)
    for o in one_offs:
        ones[pl.ds(o, 16)] = jnp.full((16,), 1.0, jnp.float32)

    plsc.subcore_barrier()

    _agg_pipeline(NCHUNK0, CH0, h_hbm, srcv, dstv, acc_sh, rows0, rows1,
                  g0, g1, s0, s1, deg_sh=deg_sh, ones=ones, dsem=dsem)

    plsc.subcore_barrier()

    _copy_out_rows(acc_sh, acc_out, cid, sid)

    @pl.when(sid < NDEG)
    def _():
        pltpu.sync_copy(deg_sh.at[pl.ds(sid * DPT, DPT)], degv)
        pltpu.sync_copy(degv, deg_out.at[pl.ds(cid * N + sid * DPT, DPT)])


_sc_agg0 = functools.partial(
    pl.kernel,
    out_type=[jax.ShapeDtypeStruct((NC, N, D_HID), jnp.float32),
              jax.ShapeDtypeStruct((NC * N,), jnp.float32)],
    mesh=_MESH,
    scratch_types=[
        pltpu.VMEM((EPT,), jnp.int32),
        pltpu.VMEM((NCHUNK0, CH0), jnp.int32),
        pltpu.VMEM((CH0, D_HID), jnp.float32),
        pltpu.VMEM((CH0, D_HID), jnp.float32),
        pltpu.VMEM((CH0,), jnp.float32),
        pltpu.VMEM((DPT,), jnp.float32),
        pltpu.VMEM_SHARED((N, D_HID), jnp.float32),
        pltpu.VMEM_SHARED((N,), jnp.float32),
        pltpu.SemaphoreType.DMA,
        pltpu.SemaphoreType.DMA,
        pltpu.SemaphoreType.DMA,
        pltpu.SemaphoreType.DMA,
        pltpu.SemaphoreType.DMA,
    ],
)(_sc_agg0_body)


def _sc_agg1_body(h_hbm, src_hbm, dst_hbm, z2d_hbm,
                  acc_out,
                  srcv, dstv, rows0, rows1, acc_sh, g0, g1, s0, s1):
    cid = lax.axis_index("c")
    sid = lax.axis_index("s")
    wid = cid * NS + sid

    _zero_init_rows(z2d_hbm, acc_sh, sid)
    pltpu.sync_copy(src_hbm.at[wid], srcv)
    pltpu.sync_copy(dst_hbm.at[wid], dstv)

    plsc.subcore_barrier()

    _agg_pipeline(NCHUNK1, CH1, h_hbm, srcv, dstv, acc_sh, rows0, rows1,
                  g0, g1, s0, s1)

    plsc.subcore_barrier()

    _copy_out_rows(acc_sh, acc_out, cid, sid)


_sc_agg1 = functools.partial(
    pl.kernel,
    out_type=jax.ShapeDtypeStruct((NC, N, N_CLASSES), jnp.float32),
    mesh=_MESH,
    scratch_types=[
        pltpu.VMEM((EPT,), jnp.int32),
        pltpu.VMEM((NCHUNK1, CH1), jnp.int32),
        pltpu.VMEM((CH1, N_CLASSES), jnp.float32),
        pltpu.VMEM((CH1, N_CLASSES), jnp.float32),
        pltpu.VMEM_SHARED((N, N_CLASSES), jnp.float32),
        pltpu.SemaphoreType.DMA,
        pltpu.SemaphoreType.DMA,
        pltpu.SemaphoreType.DMA,
        pltpu.SemaphoreType.DMA,
    ],
    compiler_params=pltpu.CompilerParams(use_tc_tiling_on_sc=False),
)(_sc_agg1_body)


def _tc1_body(x_ref, acc_ref, deg_ref, ws0_ref, wn0_ref, b0_ref,
              g0_ref, be0_ref, ws1_ref, wn1_ref, b1_ref,
              z1_ref, s1_ref, rd_ref):
    rd = 1.0 / jnp.maximum(deg_ref[0] + deg_ref[1], 1.0)
    hn = (acc_ref[0] + acc_ref[1]) * rd
    h = (jnp.dot(x_ref[...], ws0_ref[...],
                 preferred_element_type=jnp.float32)
         + jnp.dot(hn, wn0_ref[...], preferred_element_type=jnp.float32)
         + b0_ref[...])
    mu = jnp.mean(h, axis=0, keepdims=True)
    var = jnp.mean(jnp.square(h - mu), axis=0, keepdims=True)
    h = g0_ref[...] * (h - mu) * lax.rsqrt(var + 1e-5) + be0_ref[...]
    h = jnp.maximum(h, 0.0)
    z1_ref[...] = jnp.dot(h, wn1_ref[...], preferred_element_type=jnp.float32)
    s1_ref[...] = (jnp.dot(h, ws1_ref[...], preferred_element_type=jnp.float32)
                   + b1_ref[...])
    rd_ref[...] = rd


def _tc2_body(s1_ref, acc_ref, rd_ref, out_ref):
    out_ref[...] = s1_ref[...] + (acc_ref[0] + acc_ref[1]) * rd_ref[...]


def kernel(x, edge_index, W_self0, W_neigh0, b0, gamma0, beta0,
           W_self1, W_neigh1, b1):
    src2 = edge_index[0].reshape(NW, EPT)
    dst0 = edge_index[1].reshape(NW, NCHUNK0, CH0)
    dst1 = edge_index[1].reshape(NW, NCHUNK1, CH1)
    z2d = jnp.zeros((N, D_HID), jnp.float32)

    acc0, deg = _sc_agg0(x, src2, dst0, z2d)

    z1, s1, rd = pl.pallas_call(
        _tc1_body,
        out_shape=[jax.ShapeDtypeStruct((N, N_CLASSES), jnp.float32),
                   jax.ShapeDtypeStruct((N, N_CLASSES), jnp.float32),
                   jax.ShapeDtypeStruct((N, 1), jnp.float32)],
    )(x, acc0, deg.reshape(NC, N, 1), W_self0, W_neigh0,
      b0.reshape(1, -1), gamma0.reshape(1, -1), beta0.reshape(1, -1),
      W_self1, W_neigh1, b1.reshape(1, -1))

    acc1 = _sc_agg1(z1, src2, dst1, jnp.zeros((N, N_CLASSES), jnp.float32))

    out = pl.pallas_call(
        _tc2_body,
        out_shape=jax.ShapeDtypeStruct((N, N_CLASSES), jnp.float32),
    )(s1, acc1, rd)
    return out


# speedup vs baseline: 13.5185x; 1.0576x over previous
import functools

import jax
import jax.numpy as jnp
from jax import lax
from jax.experimental import pallas as pl
from jax.experimental.pallas import tpu as pltpu
from jax.experimental.pallas import tpu_sc as plsc

N = 10000
E = 320000
D_IN = 128
D_HID = 128
N_CLASSES = 64

NC = 2
NS = 16
NW = NC * NS
EPT = E // NW
CH0 = 80
NCHUNK0 = EPT // CH0
CH1 = 80
NCHUNK1 = EPT // CH1
RPT = 624
RPT_LAST = N - 15 * RPT
NDEG = 5
DPT = N // NDEG

_MESH = plsc.VectorSubcoreMesh(core_axis_name="c", subcore_axis_name="s")


def _zero_init_rows(z_hbm, acc_sh, sid):
    @pl.when(sid < NS - 1)
    def _():
        s = pl.ds(sid * RPT, RPT)
        pltpu.sync_copy(z_hbm.at[s], acc_sh.at[s])

    @pl.when(sid == NS - 1)
    def _():
        s = pl.ds((NS - 1) * RPT, RPT_LAST)
        pltpu.sync_copy(z_hbm.at[s], acc_sh.at[s])


def _copy_out_rows(acc_sh, acc_out, cid, sid):
    @pl.when(sid < NS - 1)
    def _():
        pltpu.sync_copy(acc_sh.at[pl.ds(sid * RPT, RPT)],
                        acc_out.at[cid, pl.ds(sid * RPT, RPT)])

    @pl.when(sid == NS - 1)
    def _():
        pltpu.sync_copy(acc_sh.at[pl.ds((NS - 1) * RPT, RPT_LAST)],
                        acc_out.at[cid, pl.ds((NS - 1) * RPT, RPT_LAST)])


def _agg_pipeline(nchunk, ch, h_hbm, srcv, dstv, acc_sh, rows0, rows1,
                  g0, g1, s0, s1, deg_sh=None, ones=None, dsem=None):
    def gather(i, buf, sem):
        pltpu.async_copy(h_hbm.at[srcv.at[pl.ds(i * ch, ch)]], buf, sem)

    def wait_g(buf, sem):
        pltpu.make_async_copy(h_hbm.at[srcv.at[pl.ds(0, ch)]], buf,
                              sem).wait()

    def scat(i, buf, sem):
        pltpu.async_copy(buf, acc_sh.at[dstv.at[i]], sem, add=True)

    def wait_s(buf, sem):
        pltpu.make_async_copy(buf, acc_sh.at[dstv.at[0]], sem).wait()

    def deg(i):
        if deg_sh is not None:
            if not (isinstance(i, int) and i < 4):
                @pl.when(i >= 4)
                def _():
                    pltpu.make_async_copy(ones, deg_sh.at[dstv.at[0]],
                                          dsem).wait()
            pltpu.async_copy(ones, deg_sh.at[dstv.at[i]], dsem, add=True)

    pairs = (nchunk - 1) // 2

    gather(0, rows0, g0)
    gather(1, rows1, g1)
    wait_g(rows0, g0)
    deg(0)
    scat(0, rows0, s0)

    def body(j, carry):
        a = 2 * j + 1
        wait_s(rows0, s0)
        gather(a + 1, rows0, g0)
        wait_g(rows1, g1)
        deg(a)
        scat(a, rows1, s1)
        wait_s(rows1, s1)

        @pl.when(j < pairs - 1)
        def _():
            gather(a + 2, rows1, g1)

        wait_g(rows0, g0)
        deg(a + 1)
        scat(a + 1, rows0, s0)
        return carry

    lax.fori_loop(0, pairs, body, 0)
    wait_s(rows0, s0)

    if deg_sh is not None:
        for _ in range(min(4, nchunk)):
            pltpu.make_async_copy(ones, deg_sh.at[dstv.at[0]], dsem).wait()


def _sc_agg0_body(h_hbm, ei_hbm, dst_hbm, z2d_hbm,
                  acc_out, deg_out,
                  srcv, dstv, rows0, rows1, ones, degv, acc_sh, deg_sh,
                  g0, g1, s0, s1, dsem):
    cid = lax.axis_index("c")
    sid = lax.axis_index("s")
    wid = cid * NS + sid

    _zero_init_rows(z2d_hbm, acc_sh, sid)

    def fill_zero(i, carry):
        degv[pl.ds(i * 16, 16)] = jnp.zeros((16,), jnp.float32)
        return carry

    lax.fori_loop(0, DPT // 16, fill_zero, 0)

    @pl.when(sid < NDEG)
    def _():
        pltpu.sync_copy(degv, deg_sh.at[pl.ds(sid * DPT, DPT)])

    pltpu.sync_copy(ei_hbm.at[pl.ds(wid * EPT, EPT)], srcv)
    pltpu.sync_copy(dst_hbm.at[wid], dstv)
    one_offs = list(range(0, CH0 - 15, 16))
    if CH0 % 16:
        one_offs.append(CH0 - 16)
    for o in one_offs:
        ones[pl.ds(o, 16)] = jnp.full((16,), 1.0, jnp.float32)

    plsc.subcore_barrier()

    _agg_pipeline(NCHUNK0, CH0, h_hbm, srcv, dstv, acc_sh, rows0, rows1,
                  g0, g1, s0, s1, deg_sh=deg_sh, ones=ones, dsem=dsem)

    plsc.subcore_barrier()

    _copy_out_rows(acc_sh, acc_out, cid, sid)

    @pl.when(sid < NDEG)
    def _():
        pltpu.sync_copy(deg_sh.at[pl.ds(sid * DPT, DPT)], degv)
        pltpu.sync_copy(degv, deg_out.at[pl.ds(cid * N + sid * DPT, DPT)])


_sc_agg0 = functools.partial(
    pl.kernel,
    out_type=[jax.ShapeDtypeStruct((NC, N, D_HID), jnp.float32),
              jax.ShapeDtypeStruct((NC * N,), jnp.float32)],
    mesh=_MESH,
    scratch_types=[
        pltpu.VMEM((EPT,), jnp.int32),
        pltpu.VMEM((NCHUNK0, CH0), jnp.int32),
        pltpu.VMEM((CH0, D_HID), jnp.float32),
        pltpu.VMEM((CH0, D_HID), jnp.float32),
        pltpu.VMEM((CH0,), jnp.float32),
        pltpu.VMEM((DPT,), jnp.float32),
        pltpu.VMEM_SHARED((N, D_HID), jnp.float32),
        pltpu.VMEM_SHARED((N,), jnp.float32),
        pltpu.SemaphoreType.DMA,
        pltpu.SemaphoreType.DMA,
        pltpu.SemaphoreType.DMA,
        pltpu.SemaphoreType.DMA,
        pltpu.SemaphoreType.DMA,
    ],
)(_sc_agg0_body)


def _sc_agg1_body(h_hbm, ei_hbm, dst_hbm, z2d_hbm,
                  acc_out,
                  srcv, dstv, rows0, rows1, acc_sh, g0, g1, s0, s1):
    cid = lax.axis_index("c")
    sid = lax.axis_index("s")
    wid = cid * NS + sid

    _zero_init_rows(z2d_hbm, acc_sh, sid)
    pltpu.sync_copy(ei_hbm.at[pl.ds(wid * EPT, EPT)], srcv)
    pltpu.sync_copy(dst_hbm.at[wid], dstv)

    plsc.subcore_barrier()

    _agg_pipeline(NCHUNK1, CH1, h_hbm, srcv, dstv, acc_sh, rows0, rows1,
                  g0, g1, s0, s1)

    plsc.subcore_barrier()

    _copy_out_rows(acc_sh, acc_out, cid, sid)


_sc_agg1 = functools.partial(
    pl.kernel,
    out_type=jax.ShapeDtypeStruct((NC, N, N_CLASSES), jnp.float32),
    mesh=_MESH,
    scratch_types=[
        pltpu.VMEM((EPT,), jnp.int32),
        pltpu.VMEM((NCHUNK1, CH1), jnp.int32),
        pltpu.VMEM((CH1, N_CLASSES), jnp.float32),
        pltpu.VMEM((CH1, N_CLASSES), jnp.float32),
        pltpu.VMEM_SHARED((N, N_CLASSES), jnp.float32),
        pltpu.SemaphoreType.DMA,
        pltpu.SemaphoreType.DMA,
        pltpu.SemaphoreType.DMA,
        pltpu.SemaphoreType.DMA,
    ],
    compiler_params=pltpu.CompilerParams(use_tc_tiling_on_sc=False),
)(_sc_agg1_body)


def _tc1_body(x_ref, acc_ref, deg_ref, ws0_ref, wn0_ref, b0_ref,
              g0_ref, be0_ref, ws1_ref, wn1_ref, b1_ref,
              z1_ref, s1_ref):
    deg = deg_ref[pl.ds(0, N)] + deg_ref[pl.ds(N, N)]
    rd = (1.0 / jnp.maximum(deg, 1.0))[:, None]
    hn = (acc_ref[0] + acc_ref[1]) * rd
    h = (jnp.dot(x_ref[...], ws0_ref[...],
                 preferred_element_type=jnp.float32)
         + jnp.dot(hn, wn0_ref[...], preferred_element_type=jnp.float32)
         + b0_ref[...])
    mu = jnp.mean(h, axis=0, keepdims=True)
    var = jnp.mean(jnp.square(h - mu), axis=0, keepdims=True)
    h = g0_ref[...] * (h - mu) * lax.rsqrt(var + 1e-5) + be0_ref[...]
    h = jnp.maximum(h, 0.0)
    z1_ref[...] = jnp.dot(h, wn1_ref[...], preferred_element_type=jnp.float32)
    s1_ref[...] = (jnp.dot(h, ws1_ref[...], preferred_element_type=jnp.float32)
                   + b1_ref[...])


def _tc2_body(s1_ref, acc_ref, deg_ref, out_ref):
    deg = deg_ref[pl.ds(0, N)] + deg_ref[pl.ds(N, N)]
    rd = (1.0 / jnp.maximum(deg, 1.0))[:, None]
    out_ref[...] = s1_ref[...] + (acc_ref[0] + acc_ref[1]) * rd


def kernel(x, edge_index, W_self0, W_neigh0, b0, gamma0, beta0,
           W_self1, W_neigh1, b1):
    src1 = edge_index[0]
    dst3 = edge_index[1].reshape(NW, NCHUNK0, CH0)
    z2d = jnp.zeros((N, D_HID), jnp.float32)

    acc0, deg = _sc_agg0(x, src1, dst3, z2d)

    z1, s1 = pl.pallas_call(
        _tc1_body,
        out_shape=[jax.ShapeDtypeStruct((N, N_CLASSES), jnp.float32),
                   jax.ShapeDtypeStruct((N, N_CLASSES), jnp.float32)],
    )(x, acc0, deg, W_self0, W_neigh0,
      b0.reshape(1, -1), gamma0.reshape(1, -1), beta0.reshape(1, -1),
      W_self1, W_neigh1, b1.reshape(1, -1))

    acc1 = _sc_agg1(z1, src1, dst3, jnp.zeros((N, N_CLASSES), jnp.float32))

    out = pl.pallas_call(
        _tc2_body,
        out_shape=jax.ShapeDtypeStruct((N, N_CLASSES), jnp.float32),
    )(s1, acc1, deg)
    return out


# speedup vs baseline: 13.6755x; 1.0116x over previous
import functools

import jax
import jax.numpy as jnp
from jax import lax
from jax.experimental import pallas as pl
from jax.experimental.pallas import tpu as pltpu
from jax.experimental.pallas import tpu_sc as plsc

N = 10000
E = 320000
D_IN = 128
D_HID = 128
N_CLASSES = 64

NC = 2
NS = 16
NW = NC * NS
EPT = E // NW
CH0 = 80
NCHUNK0 = EPT // CH0
CH1 = 80
NCHUNK1 = EPT // CH1
RPT = 624
RPT_LAST = N - 15 * RPT
NDEG = 5
DPT = N // NDEG

_MESH = plsc.VectorSubcoreMesh(core_axis_name="c", subcore_axis_name="s")


def _zero_init_rows(z_hbm, acc_sh, sid):
    @pl.when(sid < NS - 1)
    def _():
        s = pl.ds(sid * RPT, RPT)
        pltpu.sync_copy(z_hbm.at[s], acc_sh.at[s])

    @pl.when(sid == NS - 1)
    def _():
        s = pl.ds((NS - 1) * RPT, RPT_LAST)
        pltpu.sync_copy(z_hbm.at[s], acc_sh.at[s])


def _copy_out_rows(acc_sh, acc_out, cid, sid):
    @pl.when(sid < NS - 1)
    def _():
        pltpu.sync_copy(acc_sh.at[pl.ds(sid * RPT, RPT)],
                        acc_out.at[cid, pl.ds(sid * RPT, RPT)])

    @pl.when(sid == NS - 1)
    def _():
        pltpu.sync_copy(acc_sh.at[pl.ds((NS - 1) * RPT, RPT_LAST)],
                        acc_out.at[cid, pl.ds((NS - 1) * RPT, RPT_LAST)])


def _agg_pipeline(nchunk, ch, h_hbm, srcv, dstv, acc_sh, rows0, rows1,
                  g0, g1, s0, s1, deg_sh=None, ones=None, dsem=None):
    def gather(i, buf, sem):
        pltpu.async_copy(h_hbm.at[srcv.at[pl.ds(i * ch, ch)]], buf, sem)

    def wait_g(buf, sem):
        pltpu.make_async_copy(h_hbm.at[srcv.at[pl.ds(0, ch)]], buf,
                              sem).wait()

    def scat(i, buf, sem):
        pltpu.async_copy(buf, acc_sh.at[dstv.at[pl.ds(i * ch, ch)]], sem,
                         add=True)

    def wait_s(buf, sem):
        pltpu.make_async_copy(buf, acc_sh.at[dstv.at[pl.ds(0, ch)]],
                              sem).wait()

    def deg(i):
        if deg_sh is not None:
            if not (isinstance(i, int) and i < 4):
                @pl.when(i >= 4)
                def _():
                    pltpu.make_async_copy(ones, deg_sh.at[dstv.at[pl.ds(0, ch)]],
                                          dsem).wait()
            pltpu.async_copy(ones, deg_sh.at[dstv.at[pl.ds(i * ch, ch)]],
                             dsem, add=True)

    pairs = (nchunk - 1) // 2

    gather(0, rows0, g0)
    gather(1, rows1, g1)
    wait_g(rows0, g0)
    deg(0)
    scat(0, rows0, s0)

    def body(j, carry):
        a = 2 * j + 1
        wait_s(rows0, s0)
        gather(a + 1, rows0, g0)
        wait_g(rows1, g1)
        deg(a)
        scat(a, rows1, s1)
        wait_s(rows1, s1)

        @pl.when(j < pairs - 1)
        def _():
            gather(a + 2, rows1, g1)

        wait_g(rows0, g0)
        deg(a + 1)
        scat(a + 1, rows0, s0)
        return carry

    lax.fori_loop(0, pairs, body, 0)
    wait_s(rows0, s0)

    if deg_sh is not None:
        for _ in range(min(4, nchunk)):
            pltpu.make_async_copy(ones, deg_sh.at[dstv.at[pl.ds(0, ch)]],
                                  dsem).wait()


def _sc_agg0_body(h_hbm, ei_hbm, dst_hbm, z2d_hbm,
                  acc_out, deg_out,
                  srcv, dstv, rows0, rows1, ones, degv, acc_sh, deg_sh,
                  g0, g1, s0, s1, dsem):
    cid = lax.axis_index("c")
    sid = lax.axis_index("s")
    wid = cid * NS + sid

    _zero_init_rows(z2d_hbm, acc_sh, sid)

    def fill_zero(i, carry):
        degv[pl.ds(i * 16, 16)] = jnp.zeros((16,), jnp.float32)
        return carry

    lax.fori_loop(0, DPT // 16, fill_zero, 0)

    @pl.when(sid < NDEG)
    def _():
        pltpu.sync_copy(degv, deg_sh.at[pl.ds(sid * DPT, DPT)])

    pltpu.sync_copy(ei_hbm.at[pl.ds(wid * EPT, EPT)], srcv)
    pltpu.sync_copy(dst_hbm.at[pl.ds(wid * EPT, EPT)], dstv)
    one_offs = list(range(0, CH0 - 15, 16))
    if CH0 % 16:
        one_offs.append(CH0 - 16)
    for o in one_offs:
        ones[pl.ds(o, 16)] = jnp.full((16,), 1.0, jnp.float32)

    plsc.subcore_barrier()

    _agg_pipeline(NCHUNK0, CH0, h_hbm, srcv, dstv, acc_sh, rows0, rows1,
                  g0, g1, s0, s1, deg_sh=deg_sh, ones=ones, dsem=dsem)

    plsc.subcore_barrier()

    _copy_out_rows(acc_sh, acc_out, cid, sid)

    @pl.when(sid < NDEG)
    def _():
        pltpu.sync_copy(deg_sh.at[pl.ds(sid * DPT, DPT)], degv)
        pltpu.sync_copy(degv, deg_out.at[pl.ds(cid * N + sid * DPT, DPT)])


_sc_agg0 = functools.partial(
    pl.kernel,
    out_type=[jax.ShapeDtypeStruct((NC, N, D_HID), jnp.float32),
              jax.ShapeDtypeStruct((NC * N,), jnp.float32)],
    mesh=_MESH,
    scratch_types=[
        pltpu.VMEM((EPT,), jnp.int32),
        pltpu.VMEM((EPT,), jnp.int32),
        pltpu.VMEM((CH0, D_HID), jnp.float32),
        pltpu.VMEM((CH0, D_HID), jnp.float32),
        pltpu.VMEM((CH0,), jnp.float32),
        pltpu.VMEM((DPT,), jnp.float32),
        pltpu.VMEM_SHARED((N, D_HID), jnp.float32),
        pltpu.VMEM_SHARED((N,), jnp.float32),
        pltpu.SemaphoreType.DMA,
        pltpu.SemaphoreType.DMA,
        pltpu.SemaphoreType.DMA,
        pltpu.SemaphoreType.DMA,
        pltpu.SemaphoreType.DMA,
    ],
)(_sc_agg0_body)


def _sc_agg1_body(h_hbm, ei_hbm, dst_hbm, z2d_hbm,
                  acc_out,
                  srcv, dstv, rows0, rows1, acc_sh, g0, g1, s0, s1):
    cid = lax.axis_index("c")
    sid = lax.axis_index("s")
    wid = cid * NS + sid

    _zero_init_rows(z2d_hbm, acc_sh, sid)
    pltpu.sync_copy(ei_hbm.at[pl.ds(wid * EPT, EPT)], srcv)
    pltpu.sync_copy(dst_hbm.at[pl.ds(wid * EPT, EPT)], dstv)

    plsc.subcore_barrier()

    _agg_pipeline(NCHUNK1, CH1, h_hbm, srcv, dstv, acc_sh, rows0, rows1,
                  g0, g1, s0, s1)

    plsc.subcore_barrier()

    _copy_out_rows(acc_sh, acc_out, cid, sid)


_sc_agg1 = functools.partial(
    pl.kernel,
    out_type=jax.ShapeDtypeStruct((NC, N, N_CLASSES), jnp.float32),
    mesh=_MESH,
    scratch_types=[
        pltpu.VMEM((EPT,), jnp.int32),
        pltpu.VMEM((EPT,), jnp.int32),
        pltpu.VMEM((CH1, N_CLASSES), jnp.float32),
        pltpu.VMEM((CH1, N_CLASSES), jnp.float32),
        pltpu.VMEM_SHARED((N, N_CLASSES), jnp.float32),
        pltpu.SemaphoreType.DMA,
        pltpu.SemaphoreType.DMA,
        pltpu.SemaphoreType.DMA,
        pltpu.SemaphoreType.DMA,
    ],
    compiler_params=pltpu.CompilerParams(use_tc_tiling_on_sc=False),
)(_sc_agg1_body)


def _tc1_body(x_ref, acc_ref, deg_ref, ws0_ref, wn0_ref, b0_ref,
              g0_ref, be0_ref, ws1_ref, wn1_ref, b1_ref,
              z1_ref, s1_ref):
    deg = deg_ref[pl.ds(0, N)] + deg_ref[pl.ds(N, N)]
    rd = (1.0 / jnp.maximum(deg, 1.0))[:, None]
    hn = (acc_ref[0] + acc_ref[1]) * rd
    h = (jnp.dot(x_ref[...], ws0_ref[...],
                 preferred_element_type=jnp.float32)
         + jnp.dot(hn, wn0_ref[...], preferred_element_type=jnp.float32)
         + b0_ref[...])
    mu = jnp.mean(h, axis=0, keepdims=True)
    var = jnp.mean(jnp.square(h - mu), axis=0, keepdims=True)
    h = g0_ref[...] * (h - mu) * lax.rsqrt(var + 1e-5) + be0_ref[...]
    h = jnp.maximum(h, 0.0)
    z1_ref[...] = jnp.dot(h, wn1_ref[...], preferred_element_type=jnp.float32)
    s1_ref[...] = (jnp.dot(h, ws1_ref[...], preferred_element_type=jnp.float32)
                   + b1_ref[...])


def _tc2_body(s1_ref, acc_ref, deg_ref, out_ref):
    deg = deg_ref[pl.ds(0, N)] + deg_ref[pl.ds(N, N)]
    rd = (1.0 / jnp.maximum(deg, 1.0))[:, None]
    out_ref[...] = s1_ref[...] + (acc_ref[0] + acc_ref[1]) * rd


def kernel(x, edge_index, W_self0, W_neigh0, b0, gamma0, beta0,
           W_self1, W_neigh1, b1):
    src1 = edge_index[0]
    dst1 = edge_index[1]
    z2d = jnp.zeros((N, D_HID), jnp.float32)

    acc0, deg = _sc_agg0(x, src1, dst1, z2d)

    z1, s1 = pl.pallas_call(
        _tc1_body,
        out_shape=[jax.ShapeDtypeStruct((N, N_CLASSES), jnp.float32),
                   jax.ShapeDtypeStruct((N, N_CLASSES), jnp.float32)],
    )(x, acc0, deg, W_self0, W_neigh0,
      b0.reshape(1, -1), gamma0.reshape(1, -1), beta0.reshape(1, -1),
      W_self1, W_neigh1, b1.reshape(1, -1))

    acc1 = _sc_agg1(z1, src1, dst1, jnp.zeros((N, N_CLASSES), jnp.float32))

    out = pl.pallas_call(
        _tc2_body,
        out_shape=jax.ShapeDtypeStruct((N, N_CLASSES), jnp.float32),
    )(s1, acc1, deg)
    return out


# speedup vs baseline: 15.2213x; 1.1130x over previous
import functools

import jax
import jax.numpy as jnp
from jax import lax
from jax.experimental import pallas as pl
from jax.experimental.pallas import tpu as pltpu
from jax.experimental.pallas import tpu_sc as plsc

N = 10000
E = 320000
D_IN = 128
D_HID = 128
N_CLASSES = 64

NC = 2
NS = 16
NW = NC * NS
EPT = E // NW
CH0 = 80
NCHUNK0 = EPT // CH0
CH1 = 80
NCHUNK1 = EPT // CH1
RPT = 624
RPT_LAST = N - 15 * RPT
NDEG = 5
DPT = N // NDEG

_MESH = plsc.VectorSubcoreMesh(core_axis_name="c", subcore_axis_name="s")


def _zero_init_rows(z_hbm, acc_sh, sid):
    @pl.when(sid < NS - 1)
    def _():
        s = pl.ds(sid * RPT, RPT)
        pltpu.sync_copy(z_hbm.at[s], acc_sh.at[s])

    @pl.when(sid == NS - 1)
    def _():
        s = pl.ds((NS - 1) * RPT, RPT_LAST)
        pltpu.sync_copy(z_hbm.at[s], acc_sh.at[s])


def _copy_out_rows(acc_sh, acc_out, cid, sid):
    @pl.when(sid < NS - 1)
    def _():
        pltpu.sync_copy(acc_sh.at[pl.ds(sid * RPT, RPT)],
                        acc_out.at[cid, pl.ds(sid * RPT, RPT)])

    @pl.when(sid == NS - 1)
    def _():
        pltpu.sync_copy(acc_sh.at[pl.ds((NS - 1) * RPT, RPT_LAST)],
                        acc_out.at[cid, pl.ds((NS - 1) * RPT, RPT_LAST)])


def _agg_pipeline(nchunk, ch, h_hbm, srcv, dstv, acc_sh, rows0, rows1,
                  g0, g1, s0, s1, deg_sh=None, ones=None, dsem=None):
    def gather(i, buf, sem):
        pltpu.async_copy(h_hbm.at[srcv.at[pl.ds(i * ch, ch)]], buf, sem)

    def wait_g(buf, sem):
        pltpu.make_async_copy(h_hbm.at[srcv.at[pl.ds(0, ch)]], buf,
                              sem).wait()

    def scat(i, buf, sem):
        pltpu.async_copy(buf, acc_sh.at[dstv.at[pl.ds(i * ch, ch)]], sem,
                         add=True)

    def wait_s(buf, sem):
        pltpu.make_async_copy(buf, acc_sh.at[dstv.at[pl.ds(0, ch)]],
                              sem).wait()

    def deg(i):
        if deg_sh is not None:
            if not (isinstance(i, int) and i < 4):
                @pl.when(i >= 4)
                def _():
                    pltpu.make_async_copy(ones, deg_sh.at[dstv.at[pl.ds(0, ch)]],
                                          dsem).wait()
            pltpu.async_copy(ones, deg_sh.at[dstv.at[pl.ds(i * ch, ch)]],
                             dsem, add=True)

    pairs = (nchunk - 1) // 2

    gather(0, rows0, g0)
    gather(1, rows1, g1)
    wait_g(rows0, g0)
    deg(0)
    scat(0, rows0, s0)

    def body(j, carry):
        a = 2 * j + 1
        wait_s(rows0, s0)
        gather(a + 1, rows0, g0)
        wait_g(rows1, g1)
        deg(a)
        scat(a, rows1, s1)
        wait_s(rows1, s1)

        @pl.when(j < pairs - 1)
        def _():
            gather(a + 2, rows1, g1)

        wait_g(rows0, g0)
        deg(a + 1)
        scat(a + 1, rows0, s0)
        return carry

    lax.fori_loop(0, pairs, body, 0)
    wait_s(rows0, s0)

    if deg_sh is not None:
        for _ in range(min(4, nchunk)):
            pltpu.make_async_copy(ones, deg_sh.at[dstv.at[pl.ds(0, ch)]],
                                  dsem).wait()


def _agg_pipeline_n(nchunk, ch, h_hbm, srcv, dstv, acc_sh, bufs, gsems,
                    ssems):
    nb = len(bufs)

    def gather(i, b):
        pltpu.async_copy(h_hbm.at[srcv.at[pl.ds(i * ch, ch)]], bufs[b],
                         gsems[b])

    def wait_g(b):
        pltpu.make_async_copy(h_hbm.at[srcv.at[pl.ds(0, ch)]], bufs[b],
                              gsems[b]).wait()

    def scat(i, b):
        pltpu.async_copy(bufs[b], acc_sh.at[dstv.at[pl.ds(i * ch, ch)]],
                         ssems[b], add=True)

    def wait_s(b):
        pltpu.make_async_copy(bufs[b], acc_sh.at[dstv.at[pl.ds(0, ch)]],
                              ssems[b]).wait()

    for b in range(nb - 1):
        gather(b, b)

    groups = (nchunk + nb - 1) // nb

    def body(j, carry):
        for t in range(nb):
            c = j * nb + t

            @pl.when(jnp.logical_and(c >= 1, c < nchunk))
            def _():
                wait_s((t - 1) % nb)

            @pl.when(c + nb - 1 < nchunk)
            def _():
                gather(c + nb - 1, (t - 1) % nb)

            @pl.when(c < nchunk)
            def _():
                wait_g(t)
                scat(c, t)
        return carry

    lax.fori_loop(0, groups, body, 0)
    wait_s((nchunk - 1) % nb)


def _sc_agg0_body(h_hbm, ei_hbm, dst_hbm, z2d_hbm,
                  acc_out, deg_out,
                  srcv, dstv, rows0, rows1, ones, degv, acc_sh, deg_sh,
                  g0, g1, s0, s1, dsem):
    cid = lax.axis_index("c")
    sid = lax.axis_index("s")
    wid = cid * NS + sid

    _zero_init_rows(z2d_hbm, acc_sh, sid)

    def fill_zero(i, carry):
        degv[pl.ds(i * 16, 16)] = jnp.zeros((16,), jnp.float32)
        return carry

    lax.fori_loop(0, DPT // 16, fill_zero, 0)

    @pl.when(sid < NDEG)
    def _():
        pltpu.sync_copy(degv, deg_sh.at[pl.ds(sid * DPT, DPT)])

    pltpu.sync_copy(ei_hbm.at[pl.ds(wid * EPT, EPT)], srcv)
    pltpu.sync_copy(dst_hbm.at[pl.ds(wid * EPT, EPT)], dstv)
    one_offs = list(range(0, CH0 - 15, 16))
    if CH0 % 16:
        one_offs.append(CH0 - 16)
    for o in one_offs:
        ones[pl.ds(o, 16)] = jnp.full((16,), 1.0, jnp.float32)

    plsc.subcore_barrier()

    _agg_pipeline(NCHUNK0, CH0, h_hbm, srcv, dstv, acc_sh, rows0, rows1,
                  g0, g1, s0, s1, deg_sh=deg_sh, ones=ones, dsem=dsem)

    plsc.subcore_barrier()

    _copy_out_rows(acc_sh, acc_out, cid, sid)

    @pl.when(sid < NDEG)
    def _():
        pltpu.sync_copy(deg_sh.at[pl.ds(sid * DPT, DPT)], degv)
        pltpu.sync_copy(degv, deg_out.at[pl.ds(cid * N + sid * DPT, DPT)])


_sc_agg0 = functools.partial(
    pl.kernel,
    out_type=[jax.ShapeDtypeStruct((NC, N, D_HID), jnp.float32),
              jax.ShapeDtypeStruct((NC * N,), jnp.float32)],
    mesh=_MESH,
    scratch_types=[
        pltpu.VMEM((EPT,), jnp.int32),
        pltpu.VMEM((EPT,), jnp.int32),
        pltpu.VMEM((CH0, D_HID), jnp.float32),
        pltpu.VMEM((CH0, D_HID), jnp.float32),
        pltpu.VMEM((CH0,), jnp.float32),
        pltpu.VMEM((DPT,), jnp.float32),
        pltpu.VMEM_SHARED((N, D_HID), jnp.float32),
        pltpu.VMEM_SHARED((N,), jnp.float32),
        pltpu.SemaphoreType.DMA,
        pltpu.SemaphoreType.DMA,
        pltpu.SemaphoreType.DMA,
        pltpu.SemaphoreType.DMA,
        pltpu.SemaphoreType.DMA,
    ],
)(_sc_agg0_body)


def _sc_agg1_body(h_hbm, ei_hbm, dst_hbm, z2d_hbm,
                  acc_out,
                  srcv, dstv, rows0, rows1, rows2, rows3, acc_sh,
                  g0, g1, g2, g3, s0, s1, s2, s3):
    cid = lax.axis_index("c")
    sid = lax.axis_index("s")
    wid = cid * NS + sid

    _zero_init_rows(z2d_hbm, acc_sh, sid)
    pltpu.sync_copy(ei_hbm.at[pl.ds(wid * EPT, EPT)], srcv)
    pltpu.sync_copy(dst_hbm.at[pl.ds(wid * EPT, EPT)], dstv)

    plsc.subcore_barrier()

    _agg_pipeline_n(NCHUNK1, CH1, h_hbm, srcv, dstv, acc_sh,
                    [rows0, rows1, rows2, rows3],
                    [g0, g1, g2, g3], [s0, s1, s2, s3])

    plsc.subcore_barrier()

    _copy_out_rows(acc_sh, acc_out, cid, sid)


_sc_agg1 = functools.partial(
    pl.kernel,
    out_type=jax.ShapeDtypeStruct((NC, N, N_CLASSES), jnp.float32),
    mesh=_MESH,
    scratch_types=[
        pltpu.VMEM((EPT,), jnp.int32),
        pltpu.VMEM((EPT,), jnp.int32),
        pltpu.VMEM((CH1, N_CLASSES), jnp.float32),
        pltpu.VMEM((CH1, N_CLASSES), jnp.float32),
        pltpu.VMEM((CH1, N_CLASSES), jnp.float32),
        pltpu.VMEM((CH1, N_CLASSES), jnp.float32),
        pltpu.VMEM_SHARED((N, N_CLASSES), jnp.float32),
        pltpu.SemaphoreType.DMA,
        pltpu.SemaphoreType.DMA,
        pltpu.SemaphoreType.DMA,
        pltpu.SemaphoreType.DMA,
        pltpu.SemaphoreType.DMA,
        pltpu.SemaphoreType.DMA,
        pltpu.SemaphoreType.DMA,
        pltpu.SemaphoreType.DMA,
    ],
    compiler_params=pltpu.CompilerParams(use_tc_tiling_on_sc=False),
)(_sc_agg1_body)


def _tc1_body(x_ref, acc_ref, deg_ref, ws0_ref, wn0_ref, b0_ref,
              g0_ref, be0_ref, ws1_ref, wn1_ref, b1_ref,
              z1_ref, s1_ref):
    deg = deg_ref[pl.ds(0, N)] + deg_ref[pl.ds(N, N)]
    rd = (1.0 / jnp.maximum(deg, 1.0))[:, None]
    hn = (acc_ref[0] + acc_ref[1]) * rd
    h = (jnp.dot(x_ref[...], ws0_ref[...],
                 preferred_element_type=jnp.float32)
         + jnp.dot(hn, wn0_ref[...], preferred_element_type=jnp.float32)
         + b0_ref[...])
    mu = jnp.mean(h, axis=0, keepdims=True)
    var = jnp.mean(jnp.square(h - mu), axis=0, keepdims=True)
    h = g0_ref[...] * (h - mu) * lax.rsqrt(var + 1e-5) + be0_ref[...]
    h = jnp.maximum(h, 0.0)
    z1_ref[...] = jnp.dot(h, wn1_ref[...], preferred_element_type=jnp.float32)
    s1_ref[...] = (jnp.dot(h, ws1_ref[...], preferred_element_type=jnp.float32)
                   + b1_ref[...])


def _tc2_body(s1_ref, acc_ref, deg_ref, out_ref):
    deg = deg_ref[pl.ds(0, N)] + deg_ref[pl.ds(N, N)]
    rd = (1.0 / jnp.maximum(deg, 1.0))[:, None]
    out_ref[...] = s1_ref[...] + (acc_ref[0] + acc_ref[1]) * rd


def kernel(x, edge_index, W_self0, W_neigh0, b0, gamma0, beta0,
           W_self1, W_neigh1, b1):
    src1 = edge_index[0]
    dst1 = edge_index[1]
    z2d = jnp.zeros((N, D_HID), jnp.float32)

    acc0, deg = _sc_agg0(x, src1, dst1, z2d)

    z1, s1 = pl.pallas_call(
        _tc1_body,
        out_shape=[jax.ShapeDtypeStruct((N, N_CLASSES), jnp.float32),
                   jax.ShapeDtypeStruct((N, N_CLASSES), jnp.float32)],
    )(x, acc0, deg, W_self0, W_neigh0,
      b0.reshape(1, -1), gamma0.reshape(1, -1), beta0.reshape(1, -1),
      W_self1, W_neigh1, b1.reshape(1, -1))

    acc1 = _sc_agg1(z1, src1, dst1, jnp.zeros((N, N_CLASSES), jnp.float32))

    out = pl.pallas_call(
        _tc2_body,
        out_shape=jax.ShapeDtypeStruct((N, N_CLASSES), jnp.float32),
    )(s1, acc1, deg)
    return out


# speedup vs baseline: 16.4942x; 1.0836x over previous
import functools

import jax
import jax.numpy as jnp
from jax import lax
from jax.experimental import pallas as pl
from jax.experimental.pallas import tpu as pltpu
from jax.experimental.pallas import tpu_sc as plsc

N = 10000
E = 320000
D_IN = 128
D_HID = 128
N_CLASSES = 64

NC = 2
NS = 16
NW = NC * NS
EPT = E // NW
CH0 = 40
NCHUNK0 = EPT // CH0
CH1 = 80
NCHUNK1 = EPT // CH1
RPT = 624
RPT_LAST = N - 15 * RPT
NDEG = 5
DPT = N // NDEG

_MESH = plsc.VectorSubcoreMesh(core_axis_name="c", subcore_axis_name="s")


def _zero_init_rows(z_hbm, acc_sh, sid):
    @pl.when(sid < NS - 1)
    def _():
        s = pl.ds(sid * RPT, RPT)
        pltpu.sync_copy(z_hbm.at[s], acc_sh.at[s])

    @pl.when(sid == NS - 1)
    def _():
        s = pl.ds((NS - 1) * RPT, RPT_LAST)
        pltpu.sync_copy(z_hbm.at[s], acc_sh.at[s])


def _copy_out_rows(acc_sh, acc_out, cid, sid):
    @pl.when(sid < NS - 1)
    def _():
        pltpu.sync_copy(acc_sh.at[pl.ds(sid * RPT, RPT)],
                        acc_out.at[cid, pl.ds(sid * RPT, RPT)])

    @pl.when(sid == NS - 1)
    def _():
        pltpu.sync_copy(acc_sh.at[pl.ds((NS - 1) * RPT, RPT_LAST)],
                        acc_out.at[cid, pl.ds((NS - 1) * RPT, RPT_LAST)])


def _agg_pipeline_n(nchunk, ch, h_hbm, srcv, dstv, acc_sh, bufs, gsems,
                    ssems, deg_sh=None, ones=None, dsem=None):
    nb = len(bufs)

    def gather(i, b):
        pltpu.async_copy(h_hbm.at[srcv.at[pl.ds(i * ch, ch)]], bufs[b],
                         gsems[b])

    def wait_g(b):
        pltpu.make_async_copy(h_hbm.at[srcv.at[pl.ds(0, ch)]], bufs[b],
                              gsems[b]).wait()

    def scat(i, b):
        pltpu.async_copy(bufs[b], acc_sh.at[dstv.at[pl.ds(i * ch, ch)]],
                         ssems[b], add=True)

    def wait_s(b):
        pltpu.make_async_copy(bufs[b], acc_sh.at[dstv.at[pl.ds(0, ch)]],
                              ssems[b]).wait()

    for b in range(nb - 1):
        gather(b, b)

    groups = (nchunk + nb - 1) // nb

    def body(j, carry):
        for t in range(nb):
            c = j * nb + t

            @pl.when(jnp.logical_and(c >= 1, c < nchunk))
            def _():
                wait_s((t - 1) % nb)

            @pl.when(c + nb - 1 < nchunk)
            def _():
                gather(c + nb - 1, (t - 1) % nb)

            @pl.when(c < nchunk)
            def _():
                wait_g(t)
                if deg_sh is not None:
                    @pl.when(c >= 4)
                    def _():
                        pltpu.make_async_copy(
                            ones, deg_sh.at[dstv.at[pl.ds(0, ch)]],
                            dsem).wait()
                    pltpu.async_copy(ones,
                                     deg_sh.at[dstv.at[pl.ds(c * ch, ch)]],
                                     dsem, add=True)
                scat(c, t)
        return carry

    lax.fori_loop(0, groups, body, 0)
    wait_s((nchunk - 1) % nb)

    if deg_sh is not None:
        for _ in range(min(4, nchunk)):
            pltpu.make_async_copy(ones, deg_sh.at[dstv.at[pl.ds(0, ch)]],
                                  dsem).wait()


def _sc_agg0_body(h_hbm, ei_hbm, dst_hbm, z2d_hbm,
                  acc_out, deg_out,
                  srcv, dstv, rows0, rows1, rows2, rows3, ones, degv,
                  acc_sh, deg_sh, g0, g1, g2, g3, s0, s1, s2, s3, dsem):
    cid = lax.axis_index("c")
    sid = lax.axis_index("s")
    wid = cid * NS + sid

    _zero_init_rows(z2d_hbm, acc_sh, sid)

    def fill_zero(i, carry):
        degv[pl.ds(i * 16, 16)] = jnp.zeros((16,), jnp.float32)
        return carry

    lax.fori_loop(0, DPT // 16, fill_zero, 0)

    @pl.when(sid < NDEG)
    def _():
        pltpu.sync_copy(degv, deg_sh.at[pl.ds(sid * DPT, DPT)])

    pltpu.sync_copy(ei_hbm.at[pl.ds(wid * EPT, EPT)], srcv)
    pltpu.sync_copy(dst_hbm.at[pl.ds(wid * EPT, EPT)], dstv)
    one_offs = list(range(0, CH0 - 15, 16))
    if CH0 % 16:
        one_offs.append(CH0 - 16)
    for o in one_offs:
        ones[pl.ds(o, 16)] = jnp.full((16,), 1.0, jnp.float32)

    plsc.subcore_barrier()

    _agg_pipeline_n(NCHUNK0, CH0, h_hbm, srcv, dstv, acc_sh,
                    [rows0, rows1, rows2, rows3],
                    [g0, g1, g2, g3], [s0, s1, s2, s3],
                    deg_sh=deg_sh, ones=ones, dsem=dsem)

    plsc.subcore_barrier()

    _copy_out_rows(acc_sh, acc_out, cid, sid)

    @pl.when(sid < NDEG)
    def _():
        pltpu.sync_copy(deg_sh.at[pl.ds(sid * DPT, DPT)], degv)
        pltpu.sync_copy(degv, deg_out.at[pl.ds(cid * N + sid * DPT, DPT)])


_sc_agg0 = functools.partial(
    pl.kernel,
    out_type=[jax.ShapeDtypeStruct((NC, N, D_HID), jnp.float32),
              jax.ShapeDtypeStruct((NC * N,), jnp.float32)],
    mesh=_MESH,
    scratch_types=[
        pltpu.VMEM((EPT,), jnp.int32),
        pltpu.VMEM((EPT,), jnp.int32),
        pltpu.VMEM((CH0, D_HID), jnp.float32),
        pltpu.VMEM((CH0, D_HID), jnp.float32),
        pltpu.VMEM((CH0, D_HID), jnp.float32),
        pltpu.VMEM((CH0, D_HID), jnp.float32),
        pltpu.VMEM((CH0,), jnp.float32),
        pltpu.VMEM((DPT,), jnp.float32),
        pltpu.VMEM_SHARED((N, D_HID), jnp.float32),
        pltpu.VMEM_SHARED((N,), jnp.float32),
        pltpu.SemaphoreType.DMA,
        pltpu.SemaphoreType.DMA,
        pltpu.SemaphoreType.DMA,
        pltpu.SemaphoreType.DMA,
        pltpu.SemaphoreType.DMA,
        pltpu.SemaphoreType.DMA,
        pltpu.SemaphoreType.DMA,
        pltpu.SemaphoreType.DMA,
        pltpu.SemaphoreType.DMA,
    ],
)(_sc_agg0_body)


def _sc_agg1_body(h_hbm, ei_hbm, dst_hbm, z2d_hbm,
                  acc_out,
                  srcv, dstv, rows0, rows1, rows2, rows3, acc_sh,
                  g0, g1, g2, g3, s0, s1, s2, s3):
    cid = lax.axis_index("c")
    sid = lax.axis_index("s")
    wid = cid * NS + sid

    _zero_init_rows(z2d_hbm, acc_sh, sid)
    pltpu.sync_copy(ei_hbm.at[pl.ds(wid * EPT, EPT)], srcv)
    pltpu.sync_copy(dst_hbm.at[pl.ds(wid * EPT, EPT)], dstv)

    plsc.subcore_barrier()

    _agg_pipeline_n(NCHUNK1, CH1, h_hbm, srcv, dstv, acc_sh,
                    [rows0, rows1, rows2, rows3],
                    [g0, g1, g2, g3], [s0, s1, s2, s3])

    plsc.subcore_barrier()

    _copy_out_rows(acc_sh, acc_out, cid, sid)


_sc_agg1 = functools.partial(
    pl.kernel,
    out_type=jax.ShapeDtypeStruct((NC, N, N_CLASSES), jnp.float32),
    mesh=_MESH,
    scratch_types=[
        pltpu.VMEM((EPT,), jnp.int32),
        pltpu.VMEM((EPT,), jnp.int32),
        pltpu.VMEM((CH1, N_CLASSES), jnp.float32),
        pltpu.VMEM((CH1, N_CLASSES), jnp.float32),
        pltpu.VMEM((CH1, N_CLASSES), jnp.float32),
        pltpu.VMEM((CH1, N_CLASSES), jnp.float32),
        pltpu.VMEM_SHARED((N, N_CLASSES), jnp.float32),
        pltpu.SemaphoreType.DMA,
        pltpu.SemaphoreType.DMA,
        pltpu.SemaphoreType.DMA,
        pltpu.SemaphoreType.DMA,
        pltpu.SemaphoreType.DMA,
        pltpu.SemaphoreType.DMA,
        pltpu.SemaphoreType.DMA,
        pltpu.SemaphoreType.DMA,
    ],
    compiler_params=pltpu.CompilerParams(use_tc_tiling_on_sc=False),
)(_sc_agg1_body)


def _tc1_body(x_ref, acc_ref, deg_ref, ws0_ref, wn0_ref, b0_ref,
              g0_ref, be0_ref, ws1_ref, wn1_ref, b1_ref,
              z1_ref, s1_ref):
    deg = deg_ref[pl.ds(0, N)] + deg_ref[pl.ds(N, N)]
    rd = (1.0 / jnp.maximum(deg, 1.0))[:, None]
    hn = (acc_ref[0] + acc_ref[1]) * rd
    h = (jnp.dot(x_ref[...], ws0_ref[...],
                 preferred_element_type=jnp.float32)
         + jnp.dot(hn, wn0_ref[...], preferred_element_type=jnp.float32)
         + b0_ref[...])
    mu = jnp.mean(h, axis=0, keepdims=True)
    var = jnp.mean(jnp.square(h - mu), axis=0, keepdims=True)
    h = g0_ref[...] * (h - mu) * lax.rsqrt(var + 1e-5) + be0_ref[...]
    h = jnp.maximum(h, 0.0)
    z1_ref[...] = jnp.dot(h, wn1_ref[...], preferred_element_type=jnp.float32)
    s1_ref[...] = (jnp.dot(h, ws1_ref[...], preferred_element_type=jnp.float32)
                   + b1_ref[...])


def _tc2_body(s1_ref, acc_ref, deg_ref, out_ref):
    deg = deg_ref[pl.ds(0, N)] + deg_ref[pl.ds(N, N)]
    rd = (1.0 / jnp.maximum(deg, 1.0))[:, None]
    out_ref[...] = s1_ref[...] + (acc_ref[0] + acc_ref[1]) * rd


def kernel(x, edge_index, W_self0, W_neigh0, b0, gamma0, beta0,
           W_self1, W_neigh1, b1):
    src1 = edge_index[0]
    dst1 = edge_index[1]
    z2d = jnp.zeros((N, D_HID), jnp.float32)

    acc0, deg = _sc_agg0(x, src1, dst1, z2d)

    z1, s1 = pl.pallas_call(
        _tc1_body,
        out_shape=[jax.ShapeDtypeStruct((N, N_CLASSES), jnp.float32),
                   jax.ShapeDtypeStruct((N, N_CLASSES), jnp.float32)],
    )(x, acc0, deg, W_self0, W_neigh0,
      b0.reshape(1, -1), gamma0.reshape(1, -1), beta0.reshape(1, -1),
      W_self1, W_neigh1, b1.reshape(1, -1))

    acc1 = _sc_agg1(z1, src1, dst1, jnp.zeros((N, N_CLASSES), jnp.float32))

    out = pl.pallas_call(
        _tc2_body,
        out_shape=jax.ShapeDtypeStruct((N, N_CLASSES), jnp.float32),
    )(s1, acc1, deg)
    return out


# speedup vs baseline: 16.9675x; 1.0287x over previous
import functools

import jax
import jax.numpy as jnp
from jax import lax
from jax.experimental import pallas as pl
from jax.experimental.pallas import tpu as pltpu
from jax.experimental.pallas import tpu_sc as plsc

N = 10000
E = 320000
D_IN = 128
D_HID = 128
N_CLASSES = 64

NC = 2
NS = 16
NW = NC * NS
EPT = E // NW
CH0 = 40
NCHUNK0 = EPT // CH0
CH1 = 80
NCHUNK1 = EPT // CH1
RPT = 624
RPT_LAST = N - 15 * RPT
NDEG = 5
DPT = N // NDEG

_MESH = plsc.VectorSubcoreMesh(core_axis_name="c", subcore_axis_name="s")


def _zero_init_rows(z_hbm, acc_sh, sid):
    @pl.when(sid < NS - 1)
    def _():
        s = pl.ds(sid * RPT, RPT)
        pltpu.sync_copy(z_hbm.at[s], acc_sh.at[s])

    @pl.when(sid == NS - 1)
    def _():
        s = pl.ds((NS - 1) * RPT, RPT_LAST)
        pltpu.sync_copy(z_hbm.at[s], acc_sh.at[s])


def _copy_out_rows(acc_sh, acc_out, cid, sid):
    @pl.when(sid < NS - 1)
    def _():
        pltpu.sync_copy(acc_sh.at[pl.ds(sid * RPT, RPT)],
                        acc_out.at[cid, pl.ds(sid * RPT, RPT)])

    @pl.when(sid == NS - 1)
    def _():
        pltpu.sync_copy(acc_sh.at[pl.ds((NS - 1) * RPT, RPT_LAST)],
                        acc_out.at[cid, pl.ds((NS - 1) * RPT, RPT_LAST)])


def _agg_pipeline_n(nchunk, ch, h_hbm, srcv, dstv, acc_sh, bufs, gsems,
                    ssems, deg_sh=None, ones=None, dsem=None):
    nb = len(bufs)

    def gather(i, b):
        pltpu.async_copy(h_hbm.at[srcv.at[pl.ds(i * ch, ch)]], bufs[b],
                         gsems[b])

    def wait_g(b):
        pltpu.make_async_copy(h_hbm.at[srcv.at[pl.ds(0, ch)]], bufs[b],
                              gsems[b]).wait()

    def scat(i, b):
        pltpu.async_copy(bufs[b], acc_sh.at[dstv.at[pl.ds(i * ch, ch)]],
                         ssems[b], add=True)

    def wait_s(b):
        pltpu.make_async_copy(bufs[b], acc_sh.at[dstv.at[pl.ds(0, ch)]],
                              ssems[b]).wait()

    for b in range(nb - 1):
        gather(b, b)

    groups = (nchunk + nb - 1) // nb

    def body(j, carry):
        for t in range(nb):
            c = j * nb + t

            @pl.when(jnp.logical_and(c >= 1, c < nchunk))
            def _():
                wait_s((t - 1) % nb)

            @pl.when(c + nb - 1 < nchunk)
            def _():
                gather(c + nb - 1, (t - 1) % nb)

            @pl.when(c < nchunk)
            def _():
                wait_g(t)
                if deg_sh is not None:
                    @pl.when(c >= 4)
                    def _():
                        pltpu.make_async_copy(
                            ones, deg_sh.at[dstv.at[pl.ds(0, ch)]],
                            dsem).wait()
                    pltpu.async_copy(ones,
                                     deg_sh.at[dstv.at[pl.ds(c * ch, ch)]],
                                     dsem, add=True)
                scat(c, t)
        return carry

    lax.fori_loop(0, groups, body, 0)
    wait_s((nchunk - 1) % nb)

    if deg_sh is not None:
        for _ in range(min(4, nchunk)):
            pltpu.make_async_copy(ones, deg_sh.at[dstv.at[pl.ds(0, ch)]],
                                  dsem).wait()


def _sc_agg0_body(h_hbm, ei_hbm, dst_hbm, z2d_hbm,
                  acc_out, deg_out,
                  srcv, dstv, rows0, rows1, rows2, rows3, rows4, ones, degv,
                  acc_sh, deg_sh, g0, g1, g2, g3, g4, s0, s1, s2, s3, s4,
                  dsem):
    cid = lax.axis_index("c")
    sid = lax.axis_index("s")
    wid = cid * NS + sid

    _zero_init_rows(z2d_hbm, acc_sh, sid)

    def fill_zero(i, carry):
        degv[pl.ds(i * 16, 16)] = jnp.zeros((16,), jnp.float32)
        return carry

    lax.fori_loop(0, DPT // 16, fill_zero, 0)

    @pl.when(sid < NDEG)
    def _():
        pltpu.sync_copy(degv, deg_sh.at[pl.ds(sid * DPT, DPT)])

    pltpu.sync_copy(ei_hbm.at[pl.ds(wid * EPT, EPT)], srcv)
    pltpu.sync_copy(dst_hbm.at[pl.ds(wid * EPT, EPT)], dstv)
    one_offs = list(range(0, CH0 - 15, 16))
    if CH0 % 16:
        one_offs.append(CH0 - 16)
    for o in one_offs:
        ones[pl.ds(o, 16)] = jnp.full((16,), 1.0, jnp.float32)

    plsc.subcore_barrier()

    _agg_pipeline_n(NCHUNK0, CH0, h_hbm, srcv, dstv, acc_sh,
                    [rows0, rows1, rows2, rows3, rows4],
                    [g0, g1, g2, g3, g4], [s0, s1, s2, s3, s4],
                    deg_sh=deg_sh, ones=ones, dsem=dsem)

    plsc.subcore_barrier()

    _copy_out_rows(acc_sh, acc_out, cid, sid)

    @pl.when(sid < NDEG)
    def _():
        pltpu.sync_copy(deg_sh.at[pl.ds(sid * DPT, DPT)], degv)
        pltpu.sync_copy(degv, deg_out.at[pl.ds(cid * N + sid * DPT, DPT)])


_sc_agg0 = functools.partial(
    pl.kernel,
    out_type=[jax.ShapeDtypeStruct((NC, N, D_HID), jnp.float32),
              jax.ShapeDtypeStruct((NC * N,), jnp.float32)],
    mesh=_MESH,
    scratch_types=[
        pltpu.VMEM((EPT,), jnp.int32),
        pltpu.VMEM((EPT,), jnp.int32),
        pltpu.VMEM((CH0, D_HID), jnp.float32),
        pltpu.VMEM((CH0, D_HID), jnp.float32),
        pltpu.VMEM((CH0, D_HID), jnp.float32),
        pltpu.VMEM((CH0, D_HID), jnp.float32),
        pltpu.VMEM((CH0, D_HID), jnp.float32),
        pltpu.VMEM((CH0,), jnp.float32),
        pltpu.VMEM((DPT,), jnp.float32),
        pltpu.VMEM_SHARED((N, D_HID), jnp.float32),
        pltpu.VMEM_SHARED((N,), jnp.float32),
        pltpu.SemaphoreType.DMA,
        pltpu.SemaphoreType.DMA,
        pltpu.SemaphoreType.DMA,
        pltpu.SemaphoreType.DMA,
        pltpu.SemaphoreType.DMA,
        pltpu.SemaphoreType.DMA,
        pltpu.SemaphoreType.DMA,
        pltpu.SemaphoreType.DMA,
        pltpu.SemaphoreType.DMA,
        pltpu.SemaphoreType.DMA,
        pltpu.SemaphoreType.DMA,
    ],
)(_sc_agg0_body)


def _sc_agg1_body(h_hbm, ei_hbm, dst_hbm, z2d_hbm,
                  acc_out,
                  srcv, dstv, rows0, rows1, rows2, rows3, rows4, rows5,
                  acc_sh, g0, g1, g2, g3, g4, g5, s0, s1, s2, s3, s4, s5):
    cid = lax.axis_index("c")
    sid = lax.axis_index("s")
    wid = cid * NS + sid

    _zero_init_rows(z2d_hbm, acc_sh, sid)
    pltpu.sync_copy(ei_hbm.at[pl.ds(wid * EPT, EPT)], srcv)
    pltpu.sync_copy(dst_hbm.at[pl.ds(wid * EPT, EPT)], dstv)

    plsc.subcore_barrier()

    _agg_pipeline_n(NCHUNK1, CH1, h_hbm, srcv, dstv, acc_sh,
                    [rows0, rows1, rows2, rows3, rows4, rows5],
                    [g0, g1, g2, g3, g4, g5],
                    [s0, s1, s2, s3, s4, s5])

    plsc.subcore_barrier()

    _copy_out_rows(acc_sh, acc_out, cid, sid)


_sc_agg1 = functools.partial(
    pl.kernel,
    out_type=jax.ShapeDtypeStruct((NC, N, N_CLASSES), jnp.float32),
    mesh=_MESH,
    scratch_types=[
        pltpu.VMEM((EPT,), jnp.int32),
        pltpu.VMEM((EPT,), jnp.int32),
        pltpu.VMEM((CH1, N_CLASSES), jnp.float32),
        pltpu.VMEM((CH1, N_CLASSES), jnp.float32),
        pltpu.VMEM((CH1, N_CLASSES), jnp.float32),
        pltpu.VMEM((CH1, N_CLASSES), jnp.float32),
        pltpu.VMEM((CH1, N_CLASSES), jnp.float32),
        pltpu.VMEM((CH1, N_CLASSES), jnp.float32),
        pltpu.VMEM_SHARED((N, N_CLASSES), jnp.float32),
        pltpu.SemaphoreType.DMA,
        pltpu.SemaphoreType.DMA,
        pltpu.SemaphoreType.DMA,
        pltpu.SemaphoreType.DMA,
        pltpu.SemaphoreType.DMA,
        pltpu.SemaphoreType.DMA,
        pltpu.SemaphoreType.DMA,
        pltpu.SemaphoreType.DMA,
        pltpu.SemaphoreType.DMA,
        pltpu.SemaphoreType.DMA,
        pltpu.SemaphoreType.DMA,
        pltpu.SemaphoreType.DMA,
    ],
    compiler_params=pltpu.CompilerParams(use_tc_tiling_on_sc=False),
)(_sc_agg1_body)


def _tc1_body(x_ref, acc_ref, deg_ref, ws0_ref, wn0_ref, b0_ref,
              g0_ref, be0_ref, ws1_ref, wn1_ref, b1_ref,
              z1_ref, s1_ref):
    deg = deg_ref[pl.ds(0, N)] + deg_ref[pl.ds(N, N)]
    rd = (1.0 / jnp.maximum(deg, 1.0))[:, None]
    hn = (acc_ref[0] + acc_ref[1]) * rd
    h = (jnp.dot(x_ref[...], ws0_ref[...],
                 preferred_element_type=jnp.float32)
         + jnp.dot(hn, wn0_ref[...], preferred_element_type=jnp.float32)
         + b0_ref[...])
    mu = jnp.mean(h, axis=0, keepdims=True)
    var = jnp.mean(jnp.square(h - mu), axis=0, keepdims=True)
    h = g0_ref[...] * (h - mu) * lax.rsqrt(var + 1e-5) + be0_ref[...]
    h = jnp.maximum(h, 0.0)
    z1_ref[...] = jnp.dot(h, wn1_ref[...], preferred_element_type=jnp.float32)
    s1_ref[...] = (jnp.dot(h, ws1_ref[...], preferred_element_type=jnp.float32)
                   + b1_ref[...])


def _tc2_body(s1_ref, acc_ref, deg_ref, out_ref):
    deg = deg_ref[pl.ds(0, N)] + deg_ref[pl.ds(N, N)]
    rd = (1.0 / jnp.maximum(deg, 1.0))[:, None]
    out_ref[...] = s1_ref[...] + (acc_ref[0] + acc_ref[1]) * rd


def kernel(x, edge_index, W_self0, W_neigh0, b0, gamma0, beta0,
           W_self1, W_neigh1, b1):
    src1 = edge_index[0]
    dst1 = edge_index[1]
    z2d = jnp.zeros((N, D_HID), jnp.float32)

    acc0, deg = _sc_agg0(x, src1, dst1, z2d)

    z1, s1 = pl.pallas_call(
        _tc1_body,
        out_shape=[jax.ShapeDtypeStruct((N, N_CLASSES), jnp.float32),
                   jax.ShapeDtypeStruct((N, N_CLASSES), jnp.float32)],
    )(x, acc0, deg, W_self0, W_neigh0,
      b0.reshape(1, -1), gamma0.reshape(1, -1), beta0.reshape(1, -1),
      W_self1, W_neigh1, b1.reshape(1, -1))

    acc1 = _sc_agg1(z1, src1, dst1, jnp.zeros((N, N_CLASSES), jnp.float32))

    out = pl.pallas_call(
        _tc2_body,
        out_shape=jax.ShapeDtypeStruct((N, N_CLASSES), jnp.float32),
    )(s1, acc1, deg)
    return out


# speedup vs baseline: 17.0110x; 1.0026x over previous
import functools

import jax
import jax.numpy as jnp
from jax import lax
from jax.experimental import pallas as pl
from jax.experimental.pallas import tpu as pltpu
from jax.experimental.pallas import tpu_sc as plsc

N = 10000
E = 320000
D_IN = 128
D_HID = 128
N_CLASSES = 64

NC = 2
NS = 16
NW = NC * NS
EPT = E // NW
CH0 = 40
NCHUNK0 = EPT // CH0
CH1 = 80
NCHUNK1 = EPT // CH1
RPT = 624
RPT_LAST = N - 15 * RPT
NDEG = 5
DPT = N // NDEG

_MESH = plsc.VectorSubcoreMesh(core_axis_name="c", subcore_axis_name="s")


def _zero_init_rows(z_hbm, acc_sh, sid):
    @pl.when(sid < NS - 1)
    def _():
        s = pl.ds(sid * RPT, RPT)
        pltpu.sync_copy(z_hbm.at[s], acc_sh.at[s])

    @pl.when(sid == NS - 1)
    def _():
        s = pl.ds((NS - 1) * RPT, RPT_LAST)
        pltpu.sync_copy(z_hbm.at[s], acc_sh.at[s])


def _copy_out_rows(acc_sh, acc_out, cid, sid):
    @pl.when(sid < NS - 1)
    def _():
        pltpu.sync_copy(acc_sh.at[pl.ds(sid * RPT, RPT)],
                        acc_out.at[cid, pl.ds(sid * RPT, RPT)])

    @pl.when(sid == NS - 1)
    def _():
        pltpu.sync_copy(acc_sh.at[pl.ds((NS - 1) * RPT, RPT_LAST)],
                        acc_out.at[cid, pl.ds((NS - 1) * RPT, RPT_LAST)])


def _agg_pipeline_n(nchunk, ch, h_hbm, srcv, dstv, acc_sh, bufs, gsems,
                    ssems, deg_sh=None, ones=None, dsem=None):
    nb = len(bufs)

    def gather(i, b):
        pltpu.async_copy(h_hbm.at[srcv.at[pl.ds(i * ch, ch)]], bufs[b],
                         gsems[b])

    def wait_g(b):
        pltpu.make_async_copy(h_hbm.at[srcv.at[pl.ds(0, ch)]], bufs[b],
                              gsems[b]).wait()

    def scat(i, b):
        pltpu.async_copy(bufs[b], acc_sh.at[dstv.at[pl.ds(i * ch, ch)]],
                         ssems[b], add=True)

    def wait_s(b):
        pltpu.make_async_copy(bufs[b], acc_sh.at[dstv.at[pl.ds(0, ch)]],
                              ssems[b]).wait()

    for b in range(nb - 1):
        gather(b, b)

    groups = (nchunk + nb - 1) // nb

    def body(j, carry):
        for t in range(nb):
            c = j * nb + t

            @pl.when(jnp.logical_and(c >= 1, c < nchunk))
            def _():
                wait_s((t - 1) % nb)

            @pl.when(c + nb - 1 < nchunk)
            def _():
                gather(c + nb - 1, (t - 1) % nb)

            @pl.when(c < nchunk)
            def _():
                wait_g(t)
                if deg_sh is not None:
                    @pl.when(c >= 4)
                    def _():
                        pltpu.make_async_copy(
                            ones, deg_sh.at[dstv.at[pl.ds(0, ch)]],
                            dsem).wait()
                    pltpu.async_copy(ones,
                                     deg_sh.at[dstv.at[pl.ds(c * ch, ch)]],
                                     dsem, add=True)
                scat(c, t)
        return carry

    lax.fori_loop(0, groups, body, 0)
    wait_s((nchunk - 1) % nb)

    if deg_sh is not None:
        for _ in range(min(4, nchunk)):
            pltpu.make_async_copy(ones, deg_sh.at[dstv.at[pl.ds(0, ch)]],
                                  dsem).wait()


def _sc_agg0_body(h_hbm, ei_hbm, dst_hbm, z2d_hbm,
                  acc_out, deg_out,
                  srcv, dstv, rows0, rows1, rows2, rows3, rows4, ones, degv,
                  acc_sh, deg_sh, g0, g1, g2, g3, g4, s0, s1, s2, s3, s4,
                  dsem):
    cid = lax.axis_index("c")
    sid = lax.axis_index("s")
    wid = cid * NS + sid

    _zero_init_rows(z2d_hbm, acc_sh, sid)

    def fill_zero(i, carry):
        degv[pl.ds(i * 16, 16)] = jnp.zeros((16,), jnp.float32)
        return carry

    lax.fori_loop(0, DPT // 16, fill_zero, 0)

    @pl.when(sid < NDEG)
    def _():
        pltpu.sync_copy(degv, deg_sh.at[pl.ds(sid * DPT, DPT)])

    pltpu.sync_copy(ei_hbm.at[pl.ds(wid * EPT, EPT)], srcv)
    pltpu.sync_copy(dst_hbm.at[pl.ds(wid * EPT, EPT)], dstv)
    one_offs = list(range(0, CH0 - 15, 16))
    if CH0 % 16:
        one_offs.append(CH0 - 16)
    for o in one_offs:
        ones[pl.ds(o, 16)] = jnp.full((16,), 1.0, jnp.float32)

    plsc.subcore_barrier()

    _agg_pipeline_n(NCHUNK0, CH0, h_hbm, srcv, dstv, acc_sh,
                    [rows0, rows1, rows2, rows3, rows4],
                    [g0, g1, g2, g3, g4], [s0, s1, s2, s3, s4],
                    deg_sh=deg_sh, ones=ones, dsem=dsem)

    plsc.subcore_barrier()

    _copy_out_rows(acc_sh, acc_out, cid, sid)

    @pl.when(sid < NDEG)
    def _():
        pltpu.sync_copy(deg_sh.at[pl.ds(sid * DPT, DPT)], degv)
        pltpu.sync_copy(degv, deg_out.at[pl.ds(cid * N + sid * DPT, DPT)])


_sc_agg0 = functools.partial(
    pl.kernel,
    out_type=[jax.ShapeDtypeStruct((NC, N, D_HID), jnp.float32),
              jax.ShapeDtypeStruct((NC * N,), jnp.float32)],
    mesh=_MESH,
    scratch_types=[
        pltpu.VMEM((EPT,), jnp.int32),
        pltpu.VMEM((EPT,), jnp.int32),
        pltpu.VMEM((CH0, D_HID), jnp.float32),
        pltpu.VMEM((CH0, D_HID), jnp.float32),
        pltpu.VMEM((CH0, D_HID), jnp.float32),
        pltpu.VMEM((CH0, D_HID), jnp.float32),
        pltpu.VMEM((CH0, D_HID), jnp.float32),
        pltpu.VMEM((CH0,), jnp.float32),
        pltpu.VMEM((DPT,), jnp.float32),
        pltpu.VMEM_SHARED((N, D_HID), jnp.float32),
        pltpu.VMEM_SHARED((N,), jnp.float32),
        pltpu.SemaphoreType.DMA,
        pltpu.SemaphoreType.DMA,
        pltpu.SemaphoreType.DMA,
        pltpu.SemaphoreType.DMA,
        pltpu.SemaphoreType.DMA,
        pltpu.SemaphoreType.DMA,
        pltpu.SemaphoreType.DMA,
        pltpu.SemaphoreType.DMA,
        pltpu.SemaphoreType.DMA,
        pltpu.SemaphoreType.DMA,
        pltpu.SemaphoreType.DMA,
    ],
)(_sc_agg0_body)


def _sc_agg1_body(h_hbm, ei_hbm, dst_hbm, z2d_hbm,
                  acc_out,
                  srcv, dstv, rows0, rows1, rows2, rows3, rows4, rows5,
                  acc_sh, g0, g1, g2, g3, g4, g5, s0, s1, s2, s3, s4, s5):
    cid = lax.axis_index("c")
    sid = lax.axis_index("s")
    wid = cid * NS + sid

    _zero_init_rows(z2d_hbm, acc_sh, sid)
    pltpu.sync_copy(ei_hbm.at[pl.ds(wid * EPT, EPT)], srcv)
    pltpu.sync_copy(dst_hbm.at[pl.ds(wid * EPT, EPT)], dstv)

    plsc.subcore_barrier()

    _agg_pipeline_n(NCHUNK1, CH1, h_hbm, srcv, dstv, acc_sh,
                    [rows0, rows1, rows2, rows3, rows4, rows5],
                    [g0, g1, g2, g3, g4, g5],
                    [s0, s1, s2, s3, s4, s5])

    plsc.subcore_barrier()

    _copy_out_rows(acc_sh, acc_out, cid, sid)


_sc_agg1 = functools.partial(
    pl.kernel,
    out_type=jax.ShapeDtypeStruct((NC, N, N_CLASSES), jnp.float32),
    mesh=_MESH,
    scratch_types=[
        pltpu.VMEM((EPT,), jnp.int32),
        pltpu.VMEM((EPT,), jnp.int32),
        pltpu.VMEM((CH1, N_CLASSES), jnp.float32),
        pltpu.VMEM((CH1, N_CLASSES), jnp.float32),
        pltpu.VMEM((CH1, N_CLASSES), jnp.float32),
        pltpu.VMEM((CH1, N_CLASSES), jnp.float32),
        pltpu.VMEM((CH1, N_CLASSES), jnp.float32),
        pltpu.VMEM((CH1, N_CLASSES), jnp.float32),
        pltpu.VMEM_SHARED((N, N_CLASSES), jnp.float32),
        pltpu.SemaphoreType.DMA,
        pltpu.SemaphoreType.DMA,
        pltpu.SemaphoreType.DMA,
        pltpu.SemaphoreType.DMA,
        pltpu.SemaphoreType.DMA,
        pltpu.SemaphoreType.DMA,
        pltpu.SemaphoreType.DMA,
        pltpu.SemaphoreType.DMA,
        pltpu.SemaphoreType.DMA,
        pltpu.SemaphoreType.DMA,
        pltpu.SemaphoreType.DMA,
        pltpu.SemaphoreType.DMA,
    ],
    compiler_params=pltpu.CompilerParams(use_tc_tiling_on_sc=False),
)(_sc_agg1_body)


def _tca_body(x_ref, ws0_ref, b0_ref, xs_ref):
    xs_ref[...] = (jnp.dot(x_ref[...], ws0_ref[...],
                           preferred_element_type=jnp.float32)
                   + b0_ref[...])


def _tc1_body(xs_ref, acc_ref, deg_ref, wn0_ref,
              g0_ref, be0_ref, ws1_ref, wn1_ref, b1_ref,
              z1_ref, s1_ref):
    deg = deg_ref[pl.ds(0, N)] + deg_ref[pl.ds(N, N)]
    rd = (1.0 / jnp.maximum(deg, 1.0))[:, None]
    hn = (acc_ref[0] + acc_ref[1]) * rd
    h = (xs_ref[...]
         + jnp.dot(hn, wn0_ref[...], preferred_element_type=jnp.float32))
    mu = jnp.mean(h, axis=0, keepdims=True)
    var = jnp.mean(jnp.square(h - mu), axis=0, keepdims=True)
    h = g0_ref[...] * (h - mu) * lax.rsqrt(var + 1e-5) + be0_ref[...]
    h = jnp.maximum(h, 0.0)
    z1_ref[...] = jnp.dot(h, wn1_ref[...], preferred_element_type=jnp.float32)
    s1_ref[...] = (jnp.dot(h, ws1_ref[...], preferred_element_type=jnp.float32)
                   + b1_ref[...])


def _tc2_body(s1_ref, acc_ref, deg_ref, out_ref):
    deg = deg_ref[pl.ds(0, N)] + deg_ref[pl.ds(N, N)]
    rd = (1.0 / jnp.maximum(deg, 1.0))[:, None]
    out_ref[...] = s1_ref[...] + (acc_ref[0] + acc_ref[1]) * rd


def kernel(x, edge_index, W_self0, W_neigh0, b0, gamma0, beta0,
           W_self1, W_neigh1, b1):
    src1 = edge_index[0]
    dst1 = edge_index[1]
    z2d = jnp.zeros((N, D_HID), jnp.float32)

    acc0, deg = _sc_agg0(x, src1, dst1, z2d)

    xs0 = pl.pallas_call(
        _tca_body,
        out_shape=jax.ShapeDtypeStruct((N, D_HID), jnp.float32),
    )(x, W_self0, b0.reshape(1, -1))

    z1, s1 = pl.pallas_call(
        _tc1_body,
        out_shape=[jax.ShapeDtypeStruct((N, N_CLASSES), jnp.float32),
                   jax.ShapeDtypeStruct((N, N_CLASSES), jnp.float32)],
    )(xs0, acc0, deg, W_neigh0,
      gamma0.reshape(1, -1), beta0.reshape(1, -1),
      W_self1, W_neigh1, b1.reshape(1, -1))

    acc1 = _sc_agg1(z1, src1, dst1, jnp.zeros((N, N_CLASSES), jnp.float32))

    out = pl.pallas_call(
        _tc2_body,
        out_shape=jax.ShapeDtypeStruct((N, N_CLASSES), jnp.float32),
    )(s1, acc1, deg)
    return out


# speedup vs baseline: 18.0073x; 1.0586x over previous
import functools

import jax
import jax.numpy as jnp
from jax import lax
from jax.experimental import pallas as pl
from jax.experimental.pallas import tpu as pltpu
from jax.experimental.pallas import tpu_sc as plsc

N = 10000
E = 320000
D_IN = 128
D_HID = 128
N_CLASSES = 64

NC = 2
NS = 16
NW = NC * NS
EPT = E // NW
CH0 = 32
EPT0 = 9984
EPT0_HI = 10112
NCH_LO = EPT0 // CH0
CH1 = 80
NCHUNK1 = EPT // CH1
RPT = 624
RPT_LAST = N - 15 * RPT
NDEG = 5
DPT = N // NDEG

_MESH = plsc.VectorSubcoreMesh(core_axis_name="c", subcore_axis_name="s")


def _zero_init_rows(z_hbm, acc_sh, sid):
    @pl.when(sid < NS - 1)
    def _():
        s = pl.ds(sid * RPT, RPT)
        pltpu.sync_copy(z_hbm.at[s], acc_sh.at[s])

    @pl.when(sid == NS - 1)
    def _():
        s = pl.ds((NS - 1) * RPT, RPT_LAST)
        pltpu.sync_copy(z_hbm.at[s], acc_sh.at[s])


def _copy_out_rows(acc_sh, acc_out, cid, sid):
    @pl.when(sid < NS - 1)
    def _():
        pltpu.sync_copy(acc_sh.at[pl.ds(sid * RPT, RPT)],
                        acc_out.at[cid, pl.ds(sid * RPT, RPT)])

    @pl.when(sid == NS - 1)
    def _():
        pltpu.sync_copy(acc_sh.at[pl.ds((NS - 1) * RPT, RPT_LAST)],
                        acc_out.at[cid, pl.ds((NS - 1) * RPT, RPT_LAST)])


def _agg_pipeline_n(nchunk, ch, h_hbm, srcv, dstv, acc_sh, bufs, gsems,
                    ssems, deg_sh=None, ones=None, dsem=None):
    nb = len(bufs)

    def gather(i, b):
        pltpu.async_copy(h_hbm.at[srcv.at[pl.ds(i * ch, ch)]], bufs[b],
                         gsems[b])

    def wait_g(b):
        pltpu.make_async_copy(h_hbm.at[srcv.at[pl.ds(0, ch)]], bufs[b],
                              gsems[b]).wait()

    def scat(i, b):
        pltpu.async_copy(bufs[b], acc_sh.at[dstv.at[pl.ds(i * ch, ch)]],
                         ssems[b], add=True)

    def wait_s(b):
        pltpu.make_async_copy(bufs[b], acc_sh.at[dstv.at[pl.ds(0, ch)]],
                              ssems[b]).wait()

    for b in range(nb - 1):
        gather(b, b)

    groups = (nchunk + nb - 1) // nb

    def body(j, carry):
        for t in range(nb):
            c = j * nb + t

            @pl.when(jnp.logical_and(c >= 1, c < nchunk))
            def _():
                wait_s((t - 1) % nb)

            @pl.when(c + nb - 1 < nchunk)
            def _():
                gather(c + nb - 1, (t - 1) % nb)

            @pl.when(c < nchunk)
            def _():
                wait_g(t)
                if deg_sh is not None:
                    @pl.when(c >= 4)
                    def _():
                        pltpu.make_async_copy(
                            ones, deg_sh.at[dstv.at[pl.ds(0, ch)]],
                            dsem).wait()
                    pltpu.async_copy(ones,
                                     deg_sh.at[dstv.at[pl.ds(c * ch, ch)]],
                                     dsem, add=True)
                scat(c, t)
        return carry

    lax.fori_loop(0, groups, body, 0)
    if isinstance(nchunk, int):
        wait_s((nchunk - 1) % nb)
    else:
        for b in range(nb):
            @pl.when((nchunk - 1) % nb == b)
            def _():
                wait_s(b)

    if deg_sh is not None:
        for _ in range(4):
            pltpu.make_async_copy(ones, deg_sh.at[dstv.at[pl.ds(0, ch)]],
                                  dsem).wait()


def _sc_agg0_body(h_hbm, ei_hbm, z2d_hbm,
                  acc_out, deg_out,
                  eiv, rows0, rows1, rows2, rows3, rows4, ones, degv,
                  acc_sh, deg_sh, g0, g1, g2, g3, g4, s0, s1, s2, s3, s4,
                  dsem):
    cid = lax.axis_index("c")
    sid = lax.axis_index("s")
    wid = cid * NS + sid

    _zero_init_rows(z2d_hbm, acc_sh, sid)

    def fill_zero(i, carry):
        degv[pl.ds(i * 16, 16)] = jnp.zeros((16,), jnp.float32)
        return carry

    lax.fori_loop(0, DPT // 16, fill_zero, 0)

    @pl.when(sid < NDEG)
    def _():
        pltpu.sync_copy(degv, deg_sh.at[pl.ds(sid * DPT, DPT)])

    base = EPT0 * wid + 128 * jnp.maximum(wid - 28, 0)
    nchunk = NCH_LO + 4 * (wid >= 28).astype(jnp.int32)
    pltpu.sync_copy(ei_hbm.at[pl.ds(0, 2), pl.ds(base, EPT0_HI)], eiv)
    one_offs = list(range(0, CH0 - 15, 16))
    if CH0 % 16:
        one_offs.append(CH0 - 16)
    for o in one_offs:
        ones[pl.ds(o, 16)] = jnp.full((16,), 1.0, jnp.float32)

    plsc.subcore_barrier()

    _agg_pipeline_n(nchunk, CH0, h_hbm, eiv.at[0], eiv.at[1], acc_sh,
                    [rows0, rows1, rows2, rows3, rows4],
                    [g0, g1, g2, g3, g4], [s0, s1, s2, s3, s4],
                    deg_sh=deg_sh, ones=ones, dsem=dsem)

    plsc.subcore_barrier()

    _copy_out_rows(acc_sh, acc_out, cid, sid)

    @pl.when(sid < NDEG)
    def _():
        pltpu.sync_copy(deg_sh.at[pl.ds(sid * DPT, DPT)], degv)
        pltpu.sync_copy(degv, deg_out.at[pl.ds(cid * N + sid * DPT, DPT)])


_sc_agg0 = functools.partial(
    pl.kernel,
    out_type=[jax.ShapeDtypeStruct((NC, N, D_HID), jnp.float32),
              jax.ShapeDtypeStruct((NC * N,), jnp.float32)],
    mesh=_MESH,
    scratch_types=[
        pltpu.VMEM((2, EPT0_HI), jnp.int32),
        pltpu.VMEM((CH0, D_HID), jnp.float32),
        pltpu.VMEM((CH0, D_HID), jnp.float32),
        pltpu.VMEM((CH0, D_HID), jnp.float32),
        pltpu.VMEM((CH0, D_HID), jnp.float32),
        pltpu.VMEM((CH0, D_HID), jnp.float32),
        pltpu.VMEM((CH0,), jnp.float32),
        pltpu.VMEM((DPT,), jnp.float32),
        pltpu.VMEM_SHARED((N, D_HID), jnp.float32),
        pltpu.VMEM_SHARED((N,), jnp.float32),
        pltpu.SemaphoreType.DMA,
        pltpu.SemaphoreType.DMA,
        pltpu.SemaphoreType.DMA,
        pltpu.SemaphoreType.DMA,
        pltpu.SemaphoreType.DMA,
        pltpu.SemaphoreType.DMA,
        pltpu.SemaphoreType.DMA,
        pltpu.SemaphoreType.DMA,
        pltpu.SemaphoreType.DMA,
        pltpu.SemaphoreType.DMA,
        pltpu.SemaphoreType.DMA,
    ],
)(_sc_agg0_body)


def _sc_agg1_body(h_hbm, ei_hbm, dst_hbm, z2d_hbm,
                  acc_out,
                  srcv, dstv, rows0, rows1, rows2, rows3, rows4, rows5,
                  acc_sh, g0, g1, g2, g3, g4, g5, s0, s1, s2, s3, s4, s5):
    cid = lax.axis_index("c")
    sid = lax.axis_index("s")
    wid = cid * NS + sid

    _zero_init_rows(z2d_hbm, acc_sh, sid)
    pltpu.sync_copy(ei_hbm.at[pl.ds(wid * EPT, EPT)], srcv)
    pltpu.sync_copy(dst_hbm.at[pl.ds(wid * EPT, EPT)], dstv)

    plsc.subcore_barrier()

    _agg_pipeline_n(NCHUNK1, CH1, h_hbm, srcv, dstv, acc_sh,
                    [rows0, rows1, rows2, rows3, rows4, rows5],
                    [g0, g1, g2, g3, g4, g5],
                    [s0, s1, s2, s3, s4, s5])

    plsc.subcore_barrier()

    _copy_out_rows(acc_sh, acc_out, cid, sid)


_sc_agg1 = functools.partial(
    pl.kernel,
    out_type=jax.ShapeDtypeStruct((NC, N, N_CLASSES), jnp.float32),
    mesh=_MESH,
    scratch_types=[
        pltpu.VMEM((EPT,), jnp.int32),
        pltpu.VMEM((EPT,), jnp.int32),
        pltpu.VMEM((CH1, N_CLASSES), jnp.float32),
        pltpu.VMEM((CH1, N_CLASSES), jnp.float32),
        pltpu.VMEM((CH1, N_CLASSES), jnp.float32),
        pltpu.VMEM((CH1, N_CLASSES), jnp.float32),
        pltpu.VMEM((CH1, N_CLASSES), jnp.float32),
        pltpu.VMEM((CH1, N_CLASSES), jnp.float32),
        pltpu.VMEM_SHARED((N, N_CLASSES), jnp.float32),
        pltpu.SemaphoreType.DMA,
        pltpu.SemaphoreType.DMA,
        pltpu.SemaphoreType.DMA,
        pltpu.SemaphoreType.DMA,
        pltpu.SemaphoreType.DMA,
        pltpu.SemaphoreType.DMA,
        pltpu.SemaphoreType.DMA,
        pltpu.SemaphoreType.DMA,
        pltpu.SemaphoreType.DMA,
        pltpu.SemaphoreType.DMA,
        pltpu.SemaphoreType.DMA,
        pltpu.SemaphoreType.DMA,
    ],
    compiler_params=pltpu.CompilerParams(use_tc_tiling_on_sc=False),
)(_sc_agg1_body)


def _tca_body(x_ref, ws0_ref, b0_ref, xs_ref):
    xs_ref[...] = (jnp.dot(x_ref[...], ws0_ref[...],
                           preferred_element_type=jnp.float32)
                   + b0_ref[...])


def _tc1_body(xs_ref, acc_ref, deg_ref, wn0_ref,
              g0_ref, be0_ref, ws1_ref, wn1_ref, b1_ref,
              z1_ref, s1_ref):
    deg = deg_ref[pl.ds(0, N)] + deg_ref[pl.ds(N, N)]
    rd = (1.0 / jnp.maximum(deg, 1.0))[:, None]
    hn = (acc_ref[0] + acc_ref[1]) * rd
    h = (xs_ref[...]
         + jnp.dot(hn, wn0_ref[...], preferred_element_type=jnp.float32))
    mu = jnp.mean(h, axis=0, keepdims=True)
    var = jnp.mean(jnp.square(h - mu), axis=0, keepdims=True)
    h = g0_ref[...] * (h - mu) * lax.rsqrt(var + 1e-5) + be0_ref[...]
    h = jnp.maximum(h, 0.0)
    z1_ref[...] = jnp.dot(h, wn1_ref[...], preferred_element_type=jnp.float32)
    s1_ref[...] = (jnp.dot(h, ws1_ref[...], preferred_element_type=jnp.float32)
                   + b1_ref[...])


def _tc2_body(s1_ref, acc_ref, deg_ref, out_ref):
    deg = deg_ref[pl.ds(0, N)] + deg_ref[pl.ds(N, N)]
    rd = (1.0 / jnp.maximum(deg, 1.0))[:, None]
    out_ref[...] = s1_ref[...] + (acc_ref[0] + acc_ref[1]) * rd


def kernel(x, edge_index, W_self0, W_neigh0, b0, gamma0, beta0,
           W_self1, W_neigh1, b1):
    src1 = edge_index[0]
    dst1 = edge_index[1]
    z2d = jnp.zeros((N, D_HID), jnp.float32)

    acc0, deg = _sc_agg0(x, edge_index, z2d)

    xs0 = pl.pallas_call(
        _tca_body,
        out_shape=jax.ShapeDtypeStruct((N, D_HID), jnp.float32),
    )(x, W_self0, b0.reshape(1, -1))

    z1, s1 = pl.pallas_call(
        _tc1_body,
        out_shape=[jax.ShapeDtypeStruct((N, N_CLASSES), jnp.float32),
                   jax.ShapeDtypeStruct((N, N_CLASSES), jnp.float32)],
    )(xs0, acc0, deg, W_neigh0,
      gamma0.reshape(1, -1), beta0.reshape(1, -1),
      W_self1, W_neigh1, b1.reshape(1, -1))

    acc1 = _sc_agg1(z1, src1, dst1, jnp.zeros((N, N_CLASSES), jnp.float32))

    out = pl.pallas_call(
        _tc2_body,
        out_shape=jax.ShapeDtypeStruct((N, N_CLASSES), jnp.float32),
    )(s1, acc1, deg)
    return out
